# baseline (speedup 1.0000x reference)
import functools

import jax
import jax.numpy as jnp
from jax import lax
from jax.experimental import pallas as pl
from jax.experimental.pallas import tpu as pltpu
from jax.experimental.pallas import tpu_sc as plsc

F32 = jnp.float32
BF16 = jnp.bfloat16


def _pack_bf16(a):
    r, c = a.shape
    return jax.lax.bitcast_convert_type(a.reshape(r, c // 2, 2), jnp.int32)


def _unpack_bf16(a):
    r, c = a.shape
    return jax.lax.bitcast_convert_type(a, BF16).reshape(r, 2 * c)


def _ln(h, g, be):
    mu = jnp.mean(h, axis=-1, keepdims=True)
    d = h - mu
    var = jnp.mean(d * d, axis=-1, keepdims=True)
    return d * lax.rsqrt(var + 1e-5) * g + be


def _sc_gather(x, row, col):
    N, F = x.shape
    E = row.shape[0]
    NW = 32
    EW = E // NW
    K = 200
    CH = EW // K

    mesh = plsc.VectorSubcoreMesh(core_axis_name="c", subcore_axis_name="s")
    dt = x.dtype

    @functools.partial(
        pl.kernel,
        mesh=mesh,
        out_type=(jax.ShapeDtypeStruct((E, F), dt),
                  jax.ShapeDtypeStruct((E, F), dt)),
        scratch_types=[
            pltpu.VMEM((K,), jnp.int32),
            pltpu.VMEM((K,), jnp.int32),
            pltpu.VMEM((K, F), dt),
            pltpu.VMEM((K, F), dt),
            pltpu.SemaphoreType.DMA,
            pltpu.SemaphoreType.DMA,
        ],
    )
    def gk(x_hbm, row_hbm, col_hbm, src_hbm, dst_hbm,
           idx_r, idx_c, buf_r, buf_c, sem_r, sem_c):
        wid = lax.axis_index("s") * 2 + lax.axis_index("c")
        base = wid * EW

        def chunk(i, carry):
            off = base + i * K
            pltpu.sync_copy(row_hbm.at[pl.ds(off, K)], idx_r)
            pltpu.sync_copy(col_hbm.at[pl.ds(off, K)], idx_c)
            cr = pltpu.async_copy(x_hbm.at[idx_r], buf_r, sem_r)
            cc = pltpu.async_copy(x_hbm.at[idx_c], buf_c, sem_c)
            cr.wait()
            cc.wait()
            pltpu.sync_copy(buf_r, src_hbm.at[pl.ds(off, K)])
            pltpu.sync_copy(buf_c, dst_hbm.at[pl.ds(off, K)])
            return carry

        lax.fori_loop(0, CH, chunk, 0)

    return gk(x, row, col)


def _sc_counts(row, num_nodes):
    E = row.shape[0]
    NT = 16
    TE = E // NT
    K2 = _pick_chunk(TE, (400, 80, 16))
    CH = TE // K2
    HALF = num_nodes // 2
    ACC = HALF + 8
    ZCH = ACC // 8
    WCH = HALF // 8

    mesh = plsc.VectorSubcoreMesh(core_axis_name="c", subcore_axis_name="s")

    @functools.partial(
        pl.kernel, mesh=mesh,
        out_type=jax.ShapeDtypeStruct((num_nodes, 128), F32),
        scratch_types=[
            pltpu.VMEM((K2,), jnp.int32),
            pltpu.VMEM((K2,), jnp.int32),
            pltpu.VMEM((K2, 128), F32),
            pltpu.VMEM((8, 128), F32),
            pltpu.VMEM_SHARED((ACC, 128), F32),
        ],
    )
    def ck(row_hbm, cnt_hbm, rbuf, lbuf, ones_b, zbuf, cacc):
        cid = lax.axis_index("c")
        sid = lax.axis_index("s")
        nbase = cid * HALF

        zero16 = jnp.zeros((16,), F32)
        one16 = jnp.ones((16,), F32)
        for r in range(8):
            for j in range(8):
                zbuf[r, pl.ds(j * 16, 16)] = zero16

        def fill(r, carry):
            for j in range(8):
                ones_b[r, pl.ds(j * 16, 16)] = one16
            return carry

        lax.fori_loop(0, K2, fill, 0)
        nz = (ZCH + NT - 1) // NT
        for c0 in range(nz):
            g = c0 * NT + sid

            @pl.when(g < ZCH)
            def _():
                pltpu.sync_copy(zbuf, cacc.at[pl.ds(g * 8, 8)])

        plsc.subcore_barrier()

        def chunk(i, carry):
            off = sid * TE + i * K2
            pltpu.sync_copy(row_hbm.at[pl.ds(off, K2)], rbuf)
            for j in range(K2 // 16):
                v = rbuf[pl.ds(j * 16, 16)]
                lv = v - nbase
                m = (lv >= 0) & (lv < HALF)
                lbuf[pl.ds(j * 16, 16)] = jnp.where(m, lv, HALF)
            pltpu.sync_copy(ones_b, cacc.at[lbuf], add=True)
            return carry

        lax.fori_loop(0, CH, chunk, 0)
        plsc.subcore_barrier()

        nw = (WCH + NT - 1) // NT
        for c2 in range(nw):
            g = c2 * NT + sid

            @pl.when(g < WCH)
            def _():
                r0 = g * 8
                pltpu.sync_copy(cacc.at[pl.ds(r0, 8)],
                                cnt_hbm.at[pl.ds(nbase + r0, 8)])

    return ck(row)


def _pick_chunk(total, cands):
    for k in cands:
        if k <= total and total % k == 0:
            return k
    raise ValueError(f"no chunk size for {total}")


def _sc_scatter(out0, out1, row, num_nodes):
    E, HH = out0.shape
    NT = 16
    TE = E // NT
    K2 = _pick_chunk(TE, (80, 48, 16))
    CH = TE // K2
    HALF = num_nodes // 2
    ACC = HALF + 8
    ZCH = ACC // 8
    WCH = HALF // 8

    mesh = plsc.VectorSubcoreMesh(core_axis_name="c", subcore_axis_name="s")

    @functools.partial(
        pl.kernel, mesh=mesh,
        out_type=(jax.ShapeDtypeStruct((num_nodes, HH), F32),
                  jax.ShapeDtypeStruct((num_nodes, HH), F32)),
        scratch_types=[
            pltpu.VMEM((K2,), jnp.int32),
            pltpu.VMEM((K2,), jnp.int32),
            pltpu.VMEM((K2, HH), F32),
            pltpu.VMEM((K2, HH), F32),
            pltpu.VMEM((8, HH), F32),
            pltpu.VMEM_SHARED((ACC, HH), F32),
            pltpu.VMEM_SHARED((ACC, HH), F32),
        ],
    )
    def sk(o0_hbm, o1_hbm, row_hbm, s0_hbm, s1_hbm, rbuf, lbuf, dbuf0,
           dbuf1, zbuf, acc0, acc1):
        cid = lax.axis_index("c")
        sid = lax.axis_index("s")
        nbase = cid * HALF

        zero16 = jnp.zeros((16,), F32)
        for r in range(8):
            for j in range(HH // 16):
                zbuf[r, pl.ds(j * 16, 16)] = zero16
        nz = (ZCH + NT - 1) // NT
        for c0 in range(nz):
            g = c0 * NT + sid

            @pl.when(g < ZCH)
            def _():
                pltpu.sync_copy(zbuf, acc0.at[pl.ds(g * 8, 8)])
                pltpu.sync_copy(zbuf, acc1.at[pl.ds(g * 8, 8)])

        plsc.subcore_barrier()

        def chunk(i, carry):
            off = sid * TE + i * K2
            pltpu.sync_copy(row_hbm.at[pl.ds(off, K2)], rbuf)
            pltpu.sync_copy(o0_hbm.at[pl.ds(off, K2)], dbuf0)
            pltpu.sync_copy(o1_hbm.at[pl.ds(off, K2)], dbuf1)
            for j in range(K2 // 16):
                v = rbuf[pl.ds(j * 16, 16)]
                lv = v - nbase
                m = (lv >= 0) & (lv < HALF)
                lbuf[pl.ds(j * 16, 16)] = jnp.where(m, lv, HALF)
            pltpu.sync_copy(dbuf0, acc0.at[lbuf], add=True)
            pltpu.sync_copy(dbuf1, acc1.at[lbuf], add=True)
            return carry

        lax.fori_loop(0, CH, chunk, 0)
        plsc.subcore_barrier()

        nw = (WCH + NT - 1) // NT
        for c2 in range(nw):
            g = c2 * NT + sid

            @pl.when(g < WCH)
            def _():
                r0 = g * 8
                pltpu.sync_copy(acc0.at[pl.ds(r0, 8)],
                                s0_hbm.at[pl.ds(nbase + r0, 8)])
                pltpu.sync_copy(acc1.at[pl.ds(r0, 8)],
                                s1_hbm.at[pl.ds(nbase + r0, 8)])

    return sk(out0, out1, row)


def _tc_edge(src, dst, ea, w, store_ea):
    E, F = src.shape
    FE = ea.shape[1]
    H = w["w2T"].shape[0]
    BE = 512
    grid = (E // BE,)

    wlist = [w["w1sT"], w["w1dT"], w["w1eT"], w["b1"], w["g1"], w["be1"],
             w["w2T"], w["b2"], w["v1dT"], w["v1eT"], w["c1"], w["g2"],
             w["be2"], w["v2T"], w["c2"]]

    in_specs = [
        pl.BlockSpec((BE, F), lambda i: (i, 0)),
        pl.BlockSpec((BE, F), lambda i: (i, 0)),
        pl.BlockSpec((BE, FE), lambda i: (i, 0)),
    ] + [pl.BlockSpec(a.shape, lambda i: (0, 0)) for a in wlist]

    HH = H // 2
    osp = [pl.BlockSpec((BE, HH), lambda i: (i, 0)),
           pl.BlockSpec((BE, HH), lambda i: (i, 0))]
    osh = [jax.ShapeDtypeStruct((E, HH), F32),
           jax.ShapeDtypeStruct((E, HH), F32)]
    if store_ea:
        out_shape = (jax.ShapeDtypeStruct((E, H), BF16), *osh)
        out_specs = (pl.BlockSpec((BE, H), lambda i: (i, 0)), *osp)
    else:
        out_shape = tuple(osh)
        out_specs = tuple(osp)

    def body(src_ref, dst_ref, ea_ref, w1s, w1d, w1e, b1, g1, be1, w2, b2,
             v1d, v1e, c1, g2, be2, v2, c2, *outs):
        s = src_ref[...].astype(w1s.dtype)
        d = dst_ref[...]
        e = ea_ref[...].astype(w1s.dtype)
        h = jnp.dot(s, w1s[...], preferred_element_type=F32)
        h = h + jnp.dot(d.astype(w1s.dtype), w1d[...],
                        preferred_element_type=F32)
        h = h + jnp.dot(e, w1e[...], preferred_element_type=F32)
        h = _ln(jnp.maximum(h + b1[...], 0.0), g1[...], be1[...])
        mdt = w2.dtype
        ea2 = (jnp.dot(h.astype(mdt), w2[...], preferred_element_type=F32)
               + b2[...])
        ea2_b = ea2.astype(BF16)
        ea2_m = ea2.astype(mdt)
        h2 = (jnp.dot(d.astype(mdt), v1d[...], preferred_element_type=F32)
              + jnp.dot(ea2_m, v1e[...], preferred_element_type=F32))
        h2 = _ln(jnp.maximum(h2 + c1[...], 0.0), g2[...], be2[...])
        o = (jnp.dot(h2.astype(mdt), v2[...], preferred_element_type=F32)
             + c2[...])
        if store_ea:
            outs[0][...] = ea2_b
            outs[1][...] = o[:, :HH]
            outs[2][...] = o[:, HH:]
        else:
            outs[0][...] = o[:, :HH]
            outs[1][...] = o[:, HH:]

    return pl.pallas_call(
        body, grid=grid, in_specs=in_specs, out_specs=out_specs,
        out_shape=out_shape,
    )(src, dst, ea, *wlist)


def _tc_node(x, s0, s1, cnt, w):
    N, F = x.shape
    HH = s0.shape[1]
    T = w["u2T"].shape[1]
    BN = 400
    grid = (N // BN,)

    wlist = [w["u1xT"], w["u1aT"], w["c1"], w["g"], w["be"], w["u2T"],
             w["c2"]]
    in_specs = [
        pl.BlockSpec((BN, F), lambda i: (i, 0)),
        pl.BlockSpec((BN, HH), lambda i: (i, 0)),
        pl.BlockSpec((BN, HH), lambda i: (i, 0)),
        pl.BlockSpec((BN, 128), lambda i: (i, 0)),
    ] + [pl.BlockSpec(a.shape, lambda i: (0, 0)) for a in wlist]

    def body(x_ref, s0_ref, s1_ref, cnt_ref, u1x, u1a, c1, g, be, u2, c2,
             out_ref):
        mdt = u1x.dtype
        inv = 1.0 / jnp.maximum(cnt_ref[:, 0:1], 1.0)
        agg = jnp.concatenate([s0_ref[...], s1_ref[...]], axis=1) * inv
        h = (jnp.dot(x_ref[...].astype(mdt), u1x[...],
                     preferred_element_type=F32)
             + jnp.dot(agg.astype(mdt), u1a[...],
                       preferred_element_type=F32))
        h = _ln(jnp.maximum(h + c1[...], 0.0), g[...], be[...])
        out_ref[...] = (jnp.dot(h.astype(mdt), u2[...],
                                preferred_element_type=F32) + c2[...])

    return pl.pallas_call(
        body, grid=grid, in_specs=in_specs,
        out_specs=pl.BlockSpec((BN, T), lambda i: (i, 0)),
        out_shape=jax.ShapeDtypeStruct((N, T), F32),
    )(x, s0, s1, cnt, *wlist)


def _prep_edge(p_edge, p_node1, F, H, b):
    w1 = p_edge["w1"]
    v1 = p_node1["w1"]
    return {
        "w1sT": w1[:, :F].T.astype(b), "w1dT": w1[:, F:2 * F].T.astype(b),
        "w1eT": w1[:, 2 * F:].T.astype(b),
        "b1": p_edge["b1"][None, :], "g1": p_edge["g"][None, :],
        "be1": p_edge["be"][None, :], "w2T": p_edge["w2"].T.astype(b),
        "b2": p_edge["b2"][None, :],
        "v1dT": v1[:, :F].T.astype(b), "v1eT": v1[:, F:].T.astype(b),
        "c1": p_node1["b1"][None, :], "g2": p_node1["g"][None, :],
        "be2": p_node1["be"][None, :], "v2T": p_node1["w2"].T.astype(b),
        "c2": p_node1["b2"][None, :],
    }


def _prep_node2(p, F, H, b):
    u1 = p["w1"]
    return {
        "u1xT": u1[:, :F].T.astype(b), "u1aT": u1[:, F:].T.astype(b),
        "c1": p["b1"][None, :],
        "g": p["g"][None, :], "be": p["be"][None, :],
        "u2T": p["w2"].T.astype(b),
        "c2": p["b2"][None, :],
    }


def _impl(x, edge_idx, edge_attr, params):
    row = edge_idx[0]
    col = edge_idx[1]
    N = x.shape[0]
    cnt = None
    x_b = x.astype(BF16)
    ea_b = edge_attr.astype(BF16)
    for lname in ("l1", "l2", "l3"):
        p = params[lname]
        F = x_b.shape[1]
        H = p["edge"]["w2"].shape[0]
        last = lname == "l3"
        mdt = F32
        ew = _prep_edge(p["edge"], p["node1"], F, H, mdt)
        nw = _prep_node2(p["node2"], F, H, mdt)

        if (F // 2) % 128 == 0:
            xp = _pack_bf16(x_b)
            srcp, dstp = _sc_gather(xp, row, col)
            src = _unpack_bf16(srcp)
            dst = _unpack_bf16(dstp)
        else:
            src, dst = _sc_gather(x, row, col)
        if last:
            out0, out1 = _tc_edge(src, dst, ea_b, ew, store_ea=False)
            ea_next = None
        else:
            ea_next, out0, out1 = _tc_edge(src, dst, ea_b, ew,
                                           store_ea=True)
        if cnt is None:
            cnt = _sc_counts(row, N)
        s0, s1 = _sc_scatter(out0, out1, row, N)
        x = _tc_node(x_b, s0, s1, cnt, nw)
        x_b = x.astype(BF16)
        ea_b = ea_next
    return x


kernel = jax.jit(_impl)


# speedup vs baseline: 1.6608x; 1.6608x over previous
import functools

import jax
import jax.numpy as jnp
from jax import lax
from jax.experimental import pallas as pl
from jax.experimental.pallas import tpu as pltpu
from jax.experimental.pallas import tpu_sc as plsc

F32 = jnp.float32
BF16 = jnp.bfloat16


def _pack_cols(x):
    r, c2 = x.shape
    c = c2 // 2
    xb = x.astype(BF16)
    st = jnp.stack([xb[:, :c], xb[:, c:]], axis=-1)
    return jax.lax.bitcast_convert_type(st, jnp.int32)


def _unpk(v):
    lo = lax.bitcast_convert_type(lax.shift_left(v, 16), F32)
    hi = lax.bitcast_convert_type(v & jnp.int32(-65536), F32)
    return lo, hi


def _pk(lo, hi):
    def rtn(f):
        u = lax.bitcast_convert_type(f, jnp.int32)
        return u + jnp.int32(0x7FFF) + (lax.shift_right_logical(u, 16)
                                        & jnp.int32(1))
    lo_w = lax.shift_right_logical(rtn(lo), 16)
    hi_w = rtn(hi) & jnp.int32(-65536)
    return lo_w | hi_w


def _ln(h, g, be):
    mu = jnp.mean(h, axis=-1, keepdims=True)
    d = h - mu
    var = jnp.mean(d * d, axis=-1, keepdims=True)
    return d * lax.rsqrt(var + 1e-5) * g + be


def _sc_gather(x, row, col):
    N, F = x.shape
    E = row.shape[0]
    NW = 32
    EW = E // NW
    K = 200
    CH = EW // K
    assert CH % 2 == 0 and F <= 128

    mesh = plsc.VectorSubcoreMesh(core_axis_name="c", subcore_axis_name="s")
    dt = x.dtype

    @functools.partial(
        pl.kernel,
        mesh=mesh,
        out_type=(jax.ShapeDtypeStruct((E, F), dt),
                  jax.ShapeDtypeStruct((E, F), dt)),
        scratch_types=[
            [pltpu.VMEM((K,), jnp.int32) for _ in range(2)],
            [pltpu.VMEM((K,), jnp.int32) for _ in range(2)],
            [pltpu.VMEM((K, F), dt) for _ in range(2)],
            [pltpu.VMEM((K, F), dt) for _ in range(2)],
            [pltpu.SemaphoreType.DMA for _ in range(2)],
            [pltpu.SemaphoreType.DMA for _ in range(2)],
            [pltpu.SemaphoreType.DMA for _ in range(2)],
            [pltpu.SemaphoreType.DMA for _ in range(2)],
        ],
    )
    def gk(x_hbm, row_hbm, col_hbm, src_hbm, dst_hbm,
           idx_r, idx_c, buf_r, buf_c, sem_r, sem_c, wsem_r, wsem_c):
        wid = lax.axis_index("s") * 2 + lax.axis_index("c")
        base = wid * EW

        def start(b, i):
            off = base + i * K
            pltpu.sync_copy(row_hbm.at[pl.ds(off, K)], idx_r[b])
            pltpu.sync_copy(col_hbm.at[pl.ds(off, K)], idx_c[b])
            pltpu.async_copy(x_hbm.at[idx_r[b]], buf_r[b], sem_r[b])
            pltpu.async_copy(x_hbm.at[idx_c[b]], buf_c[b], sem_c[b])

        def finish(b, i):
            off = base + i * K
            pltpu.make_async_copy(x_hbm.at[idx_r[b]], buf_r[b],
                                  sem_r[b]).wait()
            pltpu.make_async_copy(x_hbm.at[idx_c[b]], buf_c[b],
                                  sem_c[b]).wait()
            pltpu.async_copy(buf_r[b], src_hbm.at[pl.ds(off, K)], wsem_r[b])
            pltpu.async_copy(buf_c[b], dst_hbm.at[pl.ds(off, K)], wsem_c[b])

        def wait_wb(b, i):
            off = base + i * K
            pltpu.make_async_copy(buf_r[b], src_hbm.at[pl.ds(off, K)],
                                  wsem_r[b]).wait()
            pltpu.make_async_copy(buf_c[b], dst_hbm.at[pl.ds(off, K)],
                                  wsem_c[b]).wait()

        start(0, 0)
        start(1, 1)

        def body(g, carry):
            i0 = g * 2
            finish(0, i0)
            finish(1, i0 + 1)
            wait_wb(0, i0)
            wait_wb(1, i0 + 1)
            start(0, i0 + 2)
            start(1, i0 + 3)
            return carry

        lax.fori_loop(0, CH // 2 - 1, body, 0)
        finish(0, CH - 2)
        finish(1, CH - 1)
        wait_wb(0, CH - 2)
        wait_wb(1, CH - 1)

    return gk(x, row, col)


def _sc_counts(row, num_nodes):
    E = row.shape[0]
    NT = 16
    TE = E // NT
    K2 = _pick_chunk(TE, (400, 80, 16))
    CH = TE // K2
    HALF = num_nodes // 2
    ACC = HALF + 8
    ZCH = ACC // 8
    WCH = HALF // 8

    mesh = plsc.VectorSubcoreMesh(core_axis_name="c", subcore_axis_name="s")

    @functools.partial(
        pl.kernel, mesh=mesh,
        out_type=jax.ShapeDtypeStruct((num_nodes, 128), F32),
        scratch_types=[
            pltpu.VMEM((K2,), jnp.int32),
            pltpu.VMEM((K2,), jnp.int32),
            pltpu.VMEM((K2, 128), F32),
            pltpu.VMEM((8, 128), F32),
            pltpu.VMEM_SHARED((ACC, 128), F32),
        ],
    )
    def ck(row_hbm, cnt_hbm, rbuf, lbuf, ones_b, zbuf, cacc):
        cid = lax.axis_index("c")
        sid = lax.axis_index("s")
        nbase = cid * HALF

        zero16 = jnp.zeros((16,), F32)
        one16 = jnp.ones((16,), F32)
        for r in range(8):
            for j in range(8):
                zbuf[r, pl.ds(j * 16, 16)] = zero16

        def fill(r, carry):
            for j in range(8):
                ones_b[r, pl.ds(j * 16, 16)] = one16
            return carry

        lax.fori_loop(0, K2, fill, 0)
        nz = (ZCH + NT - 1) // NT
        for c0 in range(nz):
            g = c0 * NT + sid

            @pl.when(g < ZCH)
            def _():
                pltpu.sync_copy(zbuf, cacc.at[pl.ds(g * 8, 8)])

        plsc.subcore_barrier()

        def chunk(i, carry):
            off = sid * TE + i * K2
            pltpu.sync_copy(row_hbm.at[pl.ds(off, K2)], rbuf)
            for j in range(K2 // 16):
                v = rbuf[pl.ds(j * 16, 16)]
                lv = v - nbase
                m = (lv >= 0) & (lv < HALF)
                lbuf[pl.ds(j * 16, 16)] = jnp.where(m, lv, HALF)
            pltpu.sync_copy(ones_b, cacc.at[lbuf], add=True)
            return carry

        lax.fori_loop(0, CH, chunk, 0)
        plsc.subcore_barrier()

        nw = (WCH + NT - 1) // NT
        for c2 in range(nw):
            g = c2 * NT + sid

            @pl.when(g < WCH)
            def _():
                r0 = g * 8
                pltpu.sync_copy(cacc.at[pl.ds(r0, 8)],
                                cnt_hbm.at[pl.ds(nbase + r0, 8)])

    return ck(row)


def _pick_chunk(total, cands):
    for k in cands:
        if k <= total and total % k == 0:
            return k
    raise ValueError(f"no chunk size for {total}")


def _sc_scatter(out0, out1, row, num_nodes):
    E, HH = out0.shape
    NT = 16
    TE = E // NT
    K2 = _pick_chunk(TE, (80, 48, 16))
    CH = TE // K2
    HALF = num_nodes // 2
    ACC = HALF + 8
    ZCH = ACC // 8
    WCH = HALF // 8

    mesh = plsc.VectorSubcoreMesh(core_axis_name="c", subcore_axis_name="s")

    @functools.partial(
        pl.kernel, mesh=mesh,
        out_type=(jax.ShapeDtypeStruct((num_nodes, HH), F32),
                  jax.ShapeDtypeStruct((num_nodes, HH), F32)),
        scratch_types=[
            pltpu.VMEM((K2,), jnp.int32),
            pltpu.VMEM((K2,), jnp.int32),
            pltpu.VMEM((K2, HH), F32),
            pltpu.VMEM((K2, HH), F32),
            pltpu.VMEM((8, HH), F32),
            pltpu.VMEM_SHARED((ACC, HH), F32),
            pltpu.VMEM_SHARED((ACC, HH), F32),
        ],
    )
    def sk(o0_hbm, o1_hbm, row_hbm, s0_hbm, s1_hbm, rbuf, lbuf, dbuf0,
           dbuf1, zbuf, acc0, acc1):
        cid = lax.axis_index("c")
        sid = lax.axis_index("s")
        nbase = cid * HALF

        zero16 = jnp.zeros((16,), F32)
        for r in range(8):
            for j in range(HH // 16):
                zbuf[r, pl.ds(j * 16, 16)] = zero16
        nz = (ZCH + NT - 1) // NT
        for c0 in range(nz):
            g = c0 * NT + sid

            @pl.when(g < ZCH)
            def _():
                pltpu.sync_copy(zbuf, acc0.at[pl.ds(g * 8, 8)])
                pltpu.sync_copy(zbuf, acc1.at[pl.ds(g * 8, 8)])

        plsc.subcore_barrier()

        def chunk(i, carry):
            off = sid * TE + i * K2
            pltpu.sync_copy(row_hbm.at[pl.ds(off, K2)], rbuf)
            pltpu.sync_copy(o0_hbm.at[pl.ds(off, K2)], dbuf0)
            pltpu.sync_copy(o1_hbm.at[pl.ds(off, K2)], dbuf1)
            for j in range(K2 // 16):
                v = rbuf[pl.ds(j * 16, 16)]
                lv = v - nbase
                m = (lv >= 0) & (lv < HALF)
                lbuf[pl.ds(j * 16, 16)] = jnp.where(m, lv, HALF)
            pltpu.sync_copy(dbuf0, acc0.at[lbuf], add=True)
            pltpu.sync_copy(dbuf1, acc1.at[lbuf], add=True)
            return carry

        lax.fori_loop(0, CH, chunk, 0)
        plsc.subcore_barrier()

        nw = (WCH + NT - 1) // NT
        for c2 in range(nw):
            g = c2 * NT + sid

            @pl.when(g < WCH)
            def _():
                r0 = g * 8
                pltpu.sync_copy(acc0.at[pl.ds(r0, 8)],
                                s0_hbm.at[pl.ds(nbase + r0, 8)])
                pltpu.sync_copy(acc1.at[pl.ds(r0, 8)],
                                s1_hbm.at[pl.ds(nbase + r0, 8)])

    return sk(out0, out1, row)


def _tc_edge(src, dst, ea, w, store_ea, packed):
    E, FC = src.shape
    FE = ea.shape[1]
    H = w["w2T"].shape[0]
    BE = 512
    grid = (E // BE,)

    if packed:
        wmm = [w["w1sT"][:FC], w["w1sT"][FC:], w["w1dT"][:FC],
               w["w1dT"][FC:], w["w1eT"][:FE], w["w1eT"][FE:],
               w["v1dT"][:FC], w["v1dT"][FC:]]
    else:
        wmm = [w["w1sT"], w["w1dT"], w["w1eT"], w["v1dT"]]
    wrest = [w["b1"], w["g1"], w["be1"], w["w2T"], w["b2"], w["v1eT"],
             w["c1"], w["g2"], w["be2"], w["v2T"], w["c2"]]
    wlist = wmm + wrest

    in_specs = [
        pl.BlockSpec((BE, FC), lambda i: (i, 0)),
        pl.BlockSpec((BE, FC), lambda i: (i, 0)),
        pl.BlockSpec((BE, FE), lambda i: (i, 0)),
    ] + [pl.BlockSpec(a.shape, lambda i: (0, 0)) for a in wlist]

    HH = H // 2
    osp = [pl.BlockSpec((BE, HH), lambda i: (i, 0)),
           pl.BlockSpec((BE, HH), lambda i: (i, 0))]
    osh = [jax.ShapeDtypeStruct((E, HH), F32),
           jax.ShapeDtypeStruct((E, HH), F32)]
    if store_ea:
        out_shape = (jax.ShapeDtypeStruct((E, HH), jnp.int32), *osh)
        out_specs = (pl.BlockSpec((BE, HH), lambda i: (i, 0)), *osp)
    else:
        out_shape = tuple(osh)
        out_specs = tuple(osp)

    def body(src_ref, dst_ref, ea_ref, *refs):
        ws = refs[:len(wmm)]
        (b1, g1, be1, w2, b2, v1e, c1, g2, be2, v2, c2) = \
            refs[len(wmm):len(wmm) + len(wrest)]
        outs = refs[len(wmm) + len(wrest):]

        def mm(a, b):
            return jnp.dot(a, b[...], preferred_element_type=F32)

        if packed:
            slo, shi = _unpk(src_ref[...])
            dlo, dhi = _unpk(dst_ref[...])
            elo, ehi = _unpk(ea_ref[...])
            h = (mm(slo, ws[0]) + mm(shi, ws[1]) + mm(dlo, ws[2])
                 + mm(dhi, ws[3]) + mm(elo, ws[4]) + mm(ehi, ws[5]))
        else:
            h = (mm(src_ref[...], ws[0]) + mm(dst_ref[...], ws[1])
                 + mm(ea_ref[...], ws[2]))
        h = _ln(jnp.maximum(h + b1[...], 0.0), g1[...], be1[...])
        ea2 = mm(h, w2) + b2[...]
        h2 = mm(ea2, v1e)
        if packed:
            h2 = h2 + mm(dlo, ws[6]) + mm(dhi, ws[7])
        else:
            h2 = h2 + mm(dst_ref[...], ws[3])
        h2 = _ln(jnp.maximum(h2 + c1[...], 0.0), g2[...], be2[...])
        o = mm(h2, v2) + c2[...]
        if store_ea:
            outs[0][...] = _pk(ea2[:, :HH], ea2[:, HH:])
            outs[1][...] = o[:, :HH]
            outs[2][...] = o[:, HH:]
        else:
            outs[0][...] = o[:, :HH]
            outs[1][...] = o[:, HH:]

    return pl.pallas_call(
        body, grid=grid, in_specs=in_specs, out_specs=out_specs,
        out_shape=out_shape,
    )(src, dst, ea, *wlist)


def _tc_node(x, s0, s1, cnt, w):
    N, F = x.shape
    HH = s0.shape[1]
    T = w["u2T"].shape[1]
    BN = 400
    grid = (N // BN,)

    wlist = [w["u1xT"], w["u1aT"], w["c1"], w["g"], w["be"], w["u2T"],
             w["c2"]]
    in_specs = [
        pl.BlockSpec((BN, F), lambda i: (i, 0)),
        pl.BlockSpec((BN, HH), lambda i: (i, 0)),
        pl.BlockSpec((BN, HH), lambda i: (i, 0)),
        pl.BlockSpec((BN, 128), lambda i: (i, 0)),
    ] + [pl.BlockSpec(a.shape, lambda i: (0, 0)) for a in wlist]

    def body(x_ref, s0_ref, s1_ref, cnt_ref, u1x, u1a, c1, g, be, u2, c2,
             out_ref):
        inv = 1.0 / jnp.maximum(cnt_ref[:, 0:1], 1.0)
        agg = jnp.concatenate([s0_ref[...], s1_ref[...]], axis=1) * inv
        h = (jnp.dot(x_ref[...], u1x[...], preferred_element_type=F32)
             + jnp.dot(agg, u1a[...], preferred_element_type=F32))
        h = _ln(jnp.maximum(h + c1[...], 0.0), g[...], be[...])
        out_ref[...] = (jnp.dot(h, u2[...], preferred_element_type=F32)
                        + c2[...])

    return pl.pallas_call(
        body, grid=grid, in_specs=in_specs,
        out_specs=pl.BlockSpec((BN, T), lambda i: (i, 0)),
        out_shape=jax.ShapeDtypeStruct((N, T), F32),
    )(x, s0, s1, cnt, *wlist)


def _prep_edge(p_edge, p_node1, F, H):
    w1 = p_edge["w1"]
    v1 = p_node1["w1"]
    return {
        "w1sT": w1[:, :F].T, "w1dT": w1[:, F:2 * F].T,
        "w1eT": w1[:, 2 * F:].T,
        "b1": p_edge["b1"][None, :], "g1": p_edge["g"][None, :],
        "be1": p_edge["be"][None, :], "w2T": p_edge["w2"].T,
        "b2": p_edge["b2"][None, :],
        "v1dT": v1[:, :F].T, "v1eT": v1[:, F:].T,
        "c1": p_node1["b1"][None, :], "g2": p_node1["g"][None, :],
        "be2": p_node1["be"][None, :], "v2T": p_node1["w2"].T,
        "c2": p_node1["b2"][None, :],
    }


def _prep_node2(p, F, H):
    u1 = p["w1"]
    return {
        "u1xT": u1[:, :F].T, "u1aT": u1[:, F:].T,
        "c1": p["b1"][None, :],
        "g": p["g"][None, :], "be": p["be"][None, :],
        "u2T": p["w2"].T,
        "c2": p["b2"][None, :],
    }


def _impl(x, edge_idx, edge_attr, params):
    row = edge_idx[0]
    col = edge_idx[1]
    N = x.shape[0]
    cnt = None
    ea = edge_attr
    for lname in ("l1", "l2", "l3"):
        p = params[lname]
        F = x.shape[1]
        H = p["edge"]["w2"].shape[0]
        last = lname == "l3"
        packed = lname != "l1"
        ew = _prep_edge(p["edge"], p["node1"], F, H)
        nw = _prep_node2(p["node2"], F, H)

        xg = _pack_cols(x) if packed else x
        src, dst = _sc_gather(xg, row, col)
        if last:
            out0, out1 = _tc_edge(src, dst, ea, ew, store_ea=False,
                                  packed=packed)
            ea_next = None
        else:
            ea_next, out0, out1 = _tc_edge(src, dst, ea, ew, store_ea=True,
                                           packed=packed)
        if cnt is None:
            cnt = _sc_counts(row, N)
        s0, s1 = _sc_scatter(out0, out1, row, N)
        x = _tc_node(x, s0, s1, cnt, nw)
        ea = ea_next
    return x


kernel = jax.jit(_impl)


# speedup vs baseline: 2.1922x; 1.3199x over previous
import functools

import jax
import jax.numpy as jnp
from jax import lax
from jax.experimental import pallas as pl
from jax.experimental.pallas import tpu as pltpu
from jax.experimental.pallas import tpu_sc as plsc

F32 = jnp.float32
BF16 = jnp.bfloat16


def _pack_cols(x):
    r, c2 = x.shape
    c = c2 // 2
    xb = x.astype(BF16)
    st = jnp.stack([xb[:, :c], xb[:, c:]], axis=-1)
    return jax.lax.bitcast_convert_type(st, jnp.int32)


def _unpk(v):
    lo = lax.bitcast_convert_type(lax.shift_left(v, 16), F32)
    hi = lax.bitcast_convert_type(v & jnp.int32(-65536), F32)
    return lo, hi


def _pk(lo, hi):
    def rtn(f):
        u = lax.bitcast_convert_type(f, jnp.int32)
        return u + jnp.int32(0x7FFF) + (lax.shift_right_logical(u, 16)
                                        & jnp.int32(1))
    lo_w = lax.shift_right_logical(rtn(lo), 16)
    hi_w = rtn(hi) & jnp.int32(-65536)
    return lo_w | hi_w


def _ln(h, g, be):
    mu = jnp.mean(h, axis=-1, keepdims=True)
    d = h - mu
    var = jnp.mean(d * d, axis=-1, keepdims=True)
    return d * lax.rsqrt(var + 1e-5) * g + be


def _sc_gather(x, row, col):
    N, F = x.shape
    E = row.shape[0]
    NW = 32
    EW = E // NW
    K = 200
    CH = EW // K
    assert CH % 2 == 0 and F <= 128

    mesh = plsc.VectorSubcoreMesh(core_axis_name="c", subcore_axis_name="s")
    dt = x.dtype

    @functools.partial(
        pl.kernel,
        mesh=mesh,
        out_type=(jax.ShapeDtypeStruct((E, F), dt),
                  jax.ShapeDtypeStruct((E, F), dt)),
        scratch_types=[
            [pltpu.VMEM((K,), jnp.int32) for _ in range(2)],
            [pltpu.VMEM((K,), jnp.int32) for _ in range(2)],
            [pltpu.VMEM((K, F), dt) for _ in range(2)],
            [pltpu.VMEM((K, F), dt) for _ in range(2)],
            [pltpu.SemaphoreType.DMA for _ in range(2)],
            [pltpu.SemaphoreType.DMA for _ in range(2)],
            [pltpu.SemaphoreType.DMA for _ in range(2)],
            [pltpu.SemaphoreType.DMA for _ in range(2)],
        ],
    )
    def gk(x_hbm, row_hbm, col_hbm, src_hbm, dst_hbm,
           idx_r, idx_c, buf_r, buf_c, sem_r, sem_c, wsem_r, wsem_c):
        wid = lax.axis_index("s") * 2 + lax.axis_index("c")
        base = wid * EW

        def start(b, i):
            off = base + i * K
            pltpu.sync_copy(row_hbm.at[pl.ds(off, K)], idx_r[b])
            pltpu.sync_copy(col_hbm.at[pl.ds(off, K)], idx_c[b])
            pltpu.async_copy(x_hbm.at[idx_r[b]], buf_r[b], sem_r[b])
            pltpu.async_copy(x_hbm.at[idx_c[b]], buf_c[b], sem_c[b])

        def finish(b, i):
            off = base + i * K
            pltpu.make_async_copy(x_hbm.at[idx_r[b]], buf_r[b],
                                  sem_r[b]).wait()
            pltpu.make_async_copy(x_hbm.at[idx_c[b]], buf_c[b],
                                  sem_c[b]).wait()
            pltpu.async_copy(buf_r[b], src_hbm.at[pl.ds(off, K)], wsem_r[b])
            pltpu.async_copy(buf_c[b], dst_hbm.at[pl.ds(off, K)], wsem_c[b])

        def wait_wb(b, i):
            off = base + i * K
            pltpu.make_async_copy(buf_r[b], src_hbm.at[pl.ds(off, K)],
                                  wsem_r[b]).wait()
            pltpu.make_async_copy(buf_c[b], dst_hbm.at[pl.ds(off, K)],
                                  wsem_c[b]).wait()

        start(0, 0)
        start(1, 1)

        def body(g, carry):
            i0 = g * 2
            finish(0, i0)
            finish(1, i0 + 1)
            wait_wb(0, i0)
            wait_wb(1, i0 + 1)
            start(0, i0 + 2)
            start(1, i0 + 3)
            return carry

        lax.fori_loop(0, CH // 2 - 1, body, 0)
        finish(0, CH - 2)
        finish(1, CH - 1)
        wait_wb(0, CH - 2)
        wait_wb(1, CH - 1)

    return gk(x, row, col)


def _sc_counts(row, num_nodes):
    E = row.shape[0]
    NT = 16
    TE = E // NT
    K2 = _pick_chunk(TE, (400, 80, 16))
    CH = TE // K2
    HALF = num_nodes // 2
    ACC = HALF + 8
    ZCH = ACC // 8
    WCH = HALF // 8

    mesh = plsc.VectorSubcoreMesh(core_axis_name="c", subcore_axis_name="s")

    @functools.partial(
        pl.kernel, mesh=mesh,
        out_type=jax.ShapeDtypeStruct((num_nodes, 128), F32),
        scratch_types=[
            pltpu.VMEM((K2,), jnp.int32),
            pltpu.VMEM((K2,), jnp.int32),
            pltpu.VMEM((K2, 128), F32),
            pltpu.VMEM((8, 128), F32),
            pltpu.VMEM_SHARED((ACC, 128), F32),
        ],
    )
    def ck(row_hbm, cnt_hbm, rbuf, lbuf, ones_b, zbuf, cacc):
        cid = lax.axis_index("c")
        sid = lax.axis_index("s")
        nbase = cid * HALF

        zero16 = jnp.zeros((16,), F32)
        one16 = jnp.ones((16,), F32)
        for r in range(8):
            for j in range(8):
                zbuf[r, pl.ds(j * 16, 16)] = zero16

        def fill(r, carry):
            for j in range(8):
                ones_b[r, pl.ds(j * 16, 16)] = one16
            return carry

        lax.fori_loop(0, K2, fill, 0)
        nz = (ZCH + NT - 1) // NT
        for c0 in range(nz):
            g = c0 * NT + sid

            @pl.when(g < ZCH)
            def _():
                pltpu.sync_copy(zbuf, cacc.at[pl.ds(g * 8, 8)])

        plsc.subcore_barrier()

        def chunk(i, carry):
            off = sid * TE + i * K2
            pltpu.sync_copy(row_hbm.at[pl.ds(off, K2)], rbuf)
            for j in range(K2 // 16):
                v = rbuf[pl.ds(j * 16, 16)]
                lv = v - nbase
                m = (lv >= 0) & (lv < HALF)
                lbuf[pl.ds(j * 16, 16)] = jnp.where(m, lv, HALF)
            pltpu.sync_copy(ones_b, cacc.at[lbuf], add=True)
            return carry

        lax.fori_loop(0, CH, chunk, 0)
        plsc.subcore_barrier()

        nw = (WCH + NT - 1) // NT
        for c2 in range(nw):
            g = c2 * NT + sid

            @pl.when(g < WCH)
            def _():
                r0 = g * 8
                pltpu.sync_copy(cacc.at[pl.ds(r0, 8)],
                                cnt_hbm.at[pl.ds(nbase + r0, 8)])

    return ck(row)


def _pick_chunk(total, cands):
    for k in cands:
        if k <= total and total % k == 0:
            return k
    raise ValueError(f"no chunk size for {total}")


def _sc_scatter(o_stk, row, num_nodes):
    _, E, HH = o_stk.shape
    NT = 16
    TE = E // NT
    K2 = _pick_chunk(TE, (80, 48, 16))
    CH = TE // K2
    assert CH % 2 == 0
    ZR = 40 if num_nodes % 40 == 0 else 8
    ZCH = num_nodes // ZR

    mesh = plsc.VectorSubcoreMesh(core_axis_name="c", subcore_axis_name="s")

    @functools.partial(
        pl.kernel, mesh=mesh,
        out_type=jax.ShapeDtypeStruct((2, num_nodes, HH), F32),
        scratch_types=[
            [pltpu.VMEM((K2,), jnp.int32) for _ in range(2)],
            [pltpu.VMEM((K2, HH), F32) for _ in range(2)],
            pltpu.VMEM((ZR, HH), F32),
            pltpu.VMEM_SHARED((num_nodes, HH), F32),
            [pltpu.SemaphoreType.DMA for _ in range(2)],
            [pltpu.SemaphoreType.DMA for _ in range(2)],
        ],
    )
    def sk(o_hbm, row_hbm, s_hbm, rbuf, dbuf, zbuf, acc, dsem, ssem):
        cid = lax.axis_index("c")
        sid = lax.axis_index("s")

        zero16 = jnp.zeros((16,), F32)

        def zfill(r, carry):
            for j in range(HH // 16):
                zbuf[r, pl.ds(j * 16, 16)] = zero16
            return carry

        lax.fori_loop(0, ZR, zfill, 0)
        nz = (ZCH + NT - 1) // NT
        for c0 in range(nz):
            g = c0 * NT + sid

            @pl.when(g < ZCH)
            def _():
                pltpu.sync_copy(zbuf, acc.at[pl.ds(g * ZR, ZR)])

        plsc.subcore_barrier()

        def start(b, i):
            off = sid * TE + i * K2
            pltpu.sync_copy(row_hbm.at[pl.ds(off, K2)], rbuf[b])
            pltpu.async_copy(o_hbm.at[cid, pl.ds(off, K2)], dbuf[b],
                             dsem[b])

        def finish(b, i):
            off = sid * TE + i * K2
            pltpu.make_async_copy(o_hbm.at[cid, pl.ds(off, K2)], dbuf[b],
                                  dsem[b]).wait()
            pltpu.async_copy(dbuf[b], acc.at[rbuf[b]], ssem[b], add=True)

        def wait_sc(b):
            pltpu.make_async_copy(dbuf[b], acc.at[rbuf[b]], ssem[b]).wait()

        start(0, 0)
        start(1, 1)

        def body(g, carry):
            i0 = g * 2
            finish(0, i0)
            finish(1, i0 + 1)
            wait_sc(0)
            wait_sc(1)
            start(0, i0 + 2)
            start(1, i0 + 3)
            return carry

        lax.fori_loop(0, CH // 2 - 1, body, 0)
        finish(0, CH - 2)
        finish(1, CH - 1)
        wait_sc(0)
        wait_sc(1)

        plsc.subcore_barrier()

        nw = (ZCH + NT - 1) // NT
        for c2 in range(nw):
            g = c2 * NT + sid

            @pl.when(g < ZCH)
            def _():
                r0 = g * ZR
                pltpu.sync_copy(acc.at[pl.ds(r0, ZR)],
                                s_hbm.at[cid, pl.ds(r0, ZR)])

    return sk(o_stk, row)


def _tc_edge(src, dst, ea, w, store_ea, packed):
    E, FC = src.shape
    FE = ea.shape[1]
    H = w["w2T"].shape[0]
    BE = 512
    grid = (E // BE,)

    if packed:
        wmm = [w["w1sT"][:FC], w["w1sT"][FC:], w["w1dT"][:FC],
               w["w1dT"][FC:], w["w1eT"][:FE], w["w1eT"][FE:],
               w["v1dT"][:FC], w["v1dT"][FC:]]
    else:
        wmm = [w["w1sT"], w["w1dT"], w["w1eT"], w["v1dT"]]
    wrest = [w["b1"], w["g1"], w["be1"], w["w2T"], w["b2"], w["v1eT"],
             w["c1"], w["g2"], w["be2"], w["v2T"], w["c2"]]
    wlist = wmm + wrest

    in_specs = [
        pl.BlockSpec((BE, FC), lambda i: (i, 0)),
        pl.BlockSpec((BE, FC), lambda i: (i, 0)),
        pl.BlockSpec((BE, FE), lambda i: (i, 0)),
    ] + [pl.BlockSpec(a.shape, lambda i: (0, 0)) for a in wlist]

    HH = H // 2
    osp = [pl.BlockSpec((2, BE, HH), lambda i: (0, i, 0))]
    osh = [jax.ShapeDtypeStruct((2, E, HH), F32)]
    if store_ea:
        out_shape = (jax.ShapeDtypeStruct((E, HH), jnp.int32), *osh)
        out_specs = (pl.BlockSpec((BE, HH), lambda i: (i, 0)), *osp)
    else:
        out_shape = osh[0]
        out_specs = osp[0]

    def body(src_ref, dst_ref, ea_ref, *refs):
        ws = refs[:len(wmm)]
        (b1, g1, be1, w2, b2, v1e, c1, g2, be2, v2, c2) = \
            refs[len(wmm):len(wmm) + len(wrest)]
        outs = refs[len(wmm) + len(wrest):]

        def mm(a, b):
            return jnp.dot(a, b[...], preferred_element_type=F32)

        if packed:
            slo, shi = _unpk(src_ref[...])
            dlo, dhi = _unpk(dst_ref[...])
            elo, ehi = _unpk(ea_ref[...])
            h = (mm(slo, ws[0]) + mm(shi, ws[1]) + mm(dlo, ws[2])
                 + mm(dhi, ws[3]) + mm(elo, ws[4]) + mm(ehi, ws[5]))
        else:
            h = (mm(src_ref[...], ws[0]) + mm(dst_ref[...], ws[1])
                 + mm(ea_ref[...], ws[2]))
        h = _ln(jnp.maximum(h + b1[...], 0.0), g1[...], be1[...])
        ea2 = mm(h, w2) + b2[...]
        h2 = mm(ea2, v1e)
        if packed:
            h2 = h2 + mm(dlo, ws[6]) + mm(dhi, ws[7])
        else:
            h2 = h2 + mm(dst_ref[...], ws[3])
        h2 = _ln(jnp.maximum(h2 + c1[...], 0.0), g2[...], be2[...])
        o = mm(h2, v2) + c2[...]
        if store_ea:
            outs[0][...] = _pk(ea2[:, :HH], ea2[:, HH:])
            outs[1][0] = o[:, :HH]
            outs[1][1] = o[:, HH:]
        else:
            outs[0][0] = o[:, :HH]
            outs[0][1] = o[:, HH:]

    return pl.pallas_call(
        body, grid=grid, in_specs=in_specs, out_specs=out_specs,
        out_shape=out_shape,
    )(src, dst, ea, *wlist)


def _tc_node(x, s_stk, cnt, w):
    N, F = x.shape
    HH = s_stk.shape[2]
    T = w["u2T"].shape[1]
    BN = 400
    grid = (N // BN,)

    wlist = [w["u1xT"], w["u1aT"], w["c1"], w["g"], w["be"], w["u2T"],
             w["c2"]]
    in_specs = [
        pl.BlockSpec((BN, F), lambda i: (i, 0)),
        pl.BlockSpec((2, BN, HH), lambda i: (0, i, 0)),
        pl.BlockSpec((BN, 128), lambda i: (i, 0)),
    ] + [pl.BlockSpec(a.shape, lambda i: (0, 0)) for a in wlist]

    def body(x_ref, s_ref, cnt_ref, u1x, u1a, c1, g, be, u2, c2,
             out_ref):
        inv = 1.0 / jnp.maximum(cnt_ref[:, 0:1], 1.0)
        agg = jnp.concatenate([s_ref[0], s_ref[1]], axis=1) * inv
        h = (jnp.dot(x_ref[...], u1x[...], preferred_element_type=F32)
             + jnp.dot(agg, u1a[...], preferred_element_type=F32))
        h = _ln(jnp.maximum(h + c1[...], 0.0), g[...], be[...])
        out_ref[...] = (jnp.dot(h, u2[...], preferred_element_type=F32)
                        + c2[...])

    return pl.pallas_call(
        body, grid=grid, in_specs=in_specs,
        out_specs=pl.BlockSpec((BN, T), lambda i: (i, 0)),
        out_shape=jax.ShapeDtypeStruct((N, T), F32),
    )(x, s_stk, cnt, *wlist)


def _prep_edge(p_edge, p_node1, F, H):
    w1 = p_edge["w1"]
    v1 = p_node1["w1"]
    return {
        "w1sT": w1[:, :F].T, "w1dT": w1[:, F:2 * F].T,
        "w1eT": w1[:, 2 * F:].T,
        "b1": p_edge["b1"][None, :], "g1": p_edge["g"][None, :],
        "be1": p_edge["be"][None, :], "w2T": p_edge["w2"].T,
        "b2": p_edge["b2"][None, :],
        "v1dT": v1[:, :F].T, "v1eT": v1[:, F:].T,
        "c1": p_node1["b1"][None, :], "g2": p_node1["g"][None, :],
        "be2": p_node1["be"][None, :], "v2T": p_node1["w2"].T,
        "c2": p_node1["b2"][None, :],
    }


def _prep_node2(p, F, H):
    u1 = p["w1"]
    return {
        "u1xT": u1[:, :F].T, "u1aT": u1[:, F:].T,
        "c1": p["b1"][None, :],
        "g": p["g"][None, :], "be": p["be"][None, :],
        "u2T": p["w2"].T,
        "c2": p["b2"][None, :],
    }


def _impl(x, edge_idx, edge_attr, params):
    row = edge_idx[0]
    col = edge_idx[1]
    N = x.shape[0]
    cnt = None
    ea = edge_attr
    for lname in ("l1", "l2", "l3"):
        p = params[lname]
        F = x.shape[1]
        H = p["edge"]["w2"].shape[0]
        last = lname == "l3"
        packed = lname != "l1"
        ew = _prep_edge(p["edge"], p["node1"], F, H)
        nw = _prep_node2(p["node2"], F, H)

        xg = _pack_cols(x) if packed else x
        src, dst = _sc_gather(xg, row, col)
        if last:
            o_stk = _tc_edge(src, dst, ea, ew, store_ea=False,
                             packed=packed)
            ea_next = None
        else:
            ea_next, o_stk = _tc_edge(src, dst, ea, ew, store_ea=True,
                                      packed=packed)
        if cnt is None:
            cnt = _sc_counts(row, N)
        s_stk = _sc_scatter(o_stk, row, N)
        x = _tc_node(x, s_stk, cnt, nw)
        ea = ea_next
    return x


kernel = jax.jit(_impl)


# speedup vs baseline: 2.6075x; 1.1895x over previous
import functools

import jax
import jax.numpy as jnp
from jax import lax
from jax.experimental import pallas as pl
from jax.experimental.pallas import tpu as pltpu
from jax.experimental.pallas import tpu_sc as plsc

F32 = jnp.float32
BF16 = jnp.bfloat16


def _pack_cols(x):
    r, c2 = x.shape
    c = c2 // 2
    xb = x.astype(BF16)
    st = jnp.stack([xb[:, :c], xb[:, c:]], axis=-1)
    return jax.lax.bitcast_convert_type(st, jnp.int32)


def _unpk(v):
    lo = lax.bitcast_convert_type(lax.shift_left(v, 16), F32)
    hi = lax.bitcast_convert_type(v & jnp.int32(-65536), F32)
    return lo, hi


def _pk(lo, hi):
    def rtn(f):
        u = lax.bitcast_convert_type(f, jnp.int32)
        return u + jnp.int32(0x7FFF) + (lax.shift_right_logical(u, 16)
                                        & jnp.int32(1))
    lo_w = lax.shift_right_logical(rtn(lo), 16)
    hi_w = rtn(hi) & jnp.int32(-65536)
    return lo_w | hi_w


def _ln(h, g, be):
    mu = jnp.mean(h, axis=-1, keepdims=True)
    d = h - mu
    var = jnp.mean(d * d, axis=-1, keepdims=True)
    return d * lax.rsqrt(var + 1e-5) * g + be


def _sc_gather(x, row, col):
    N, F = x.shape
    E = row.shape[0]
    NW = 32
    EW = E // NW
    K = 200
    CH = EW // K
    assert CH % 2 == 0 and F <= 128

    mesh = plsc.VectorSubcoreMesh(core_axis_name="c", subcore_axis_name="s")
    dt = x.dtype

    @functools.partial(
        pl.kernel,
        mesh=mesh,
        out_type=(jax.ShapeDtypeStruct((E, F), dt),
                  jax.ShapeDtypeStruct((E, F), dt)),
        scratch_types=[
            [pltpu.VMEM((K,), jnp.int32) for _ in range(2)],
            [pltpu.VMEM((K,), jnp.int32) for _ in range(2)],
            [pltpu.VMEM((K, F), dt) for _ in range(2)],
            [pltpu.VMEM((K, F), dt) for _ in range(2)],
            [pltpu.SemaphoreType.DMA for _ in range(2)],
            [pltpu.SemaphoreType.DMA for _ in range(2)],
            [pltpu.SemaphoreType.DMA for _ in range(2)],
            [pltpu.SemaphoreType.DMA for _ in range(2)],
        ],
    )
    def gk(x_hbm, row_hbm, col_hbm, src_hbm, dst_hbm,
           idx_r, idx_c, buf_r, buf_c, sem_r, sem_c, wsem_r, wsem_c):
        wid = lax.axis_index("s") * 2 + lax.axis_index("c")
        base = wid * EW

        def start(b, i):
            off = base + i * K
            pltpu.sync_copy(row_hbm.at[pl.ds(off, K)], idx_r[b])
            pltpu.sync_copy(col_hbm.at[pl.ds(off, K)], idx_c[b])
            pltpu.async_copy(x_hbm.at[idx_r[b]], buf_r[b], sem_r[b])
            pltpu.async_copy(x_hbm.at[idx_c[b]], buf_c[b], sem_c[b])

        def finish(b, i):
            off = base + i * K
            pltpu.make_async_copy(x_hbm.at[idx_r[b]], buf_r[b],
                                  sem_r[b]).wait()
            pltpu.make_async_copy(x_hbm.at[idx_c[b]], buf_c[b],
                                  sem_c[b]).wait()
            pltpu.async_copy(buf_r[b], src_hbm.at[pl.ds(off, K)], wsem_r[b])
            pltpu.async_copy(buf_c[b], dst_hbm.at[pl.ds(off, K)], wsem_c[b])

        def wait_wb(b, i):
            off = base + i * K
            pltpu.make_async_copy(buf_r[b], src_hbm.at[pl.ds(off, K)],
                                  wsem_r[b]).wait()
            pltpu.make_async_copy(buf_c[b], dst_hbm.at[pl.ds(off, K)],
                                  wsem_c[b]).wait()

        start(0, 0)
        start(1, 1)

        def body(g, carry):
            i0 = g * 2
            finish(0, i0)
            finish(1, i0 + 1)
            wait_wb(0, i0)
            wait_wb(1, i0 + 1)
            start(0, i0 + 2)
            start(1, i0 + 3)
            return carry

        lax.fori_loop(0, CH // 2 - 1, body, 0)
        finish(0, CH - 2)
        finish(1, CH - 1)
        wait_wb(0, CH - 2)
        wait_wb(1, CH - 1)

    return gk(x, row, col)


def _sc_counts(row, num_nodes):
    E = row.shape[0]
    NT = 16
    TE = E // NT
    K2 = _pick_chunk(TE, (400, 80, 16))
    CH = TE // K2
    HALF = num_nodes // 2
    ACC = HALF + 8
    ZCH = ACC // 8
    WCH = HALF // 8

    mesh = plsc.VectorSubcoreMesh(core_axis_name="c", subcore_axis_name="s")

    @functools.partial(
        pl.kernel, mesh=mesh,
        out_type=jax.ShapeDtypeStruct((num_nodes, 128), F32),
        scratch_types=[
            pltpu.VMEM((K2,), jnp.int32),
            pltpu.VMEM((K2,), jnp.int32),
            pltpu.VMEM((K2, 128), F32),
            pltpu.VMEM((8, 128), F32),
            pltpu.VMEM_SHARED((ACC, 128), F32),
        ],
    )
    def ck(row_hbm, cnt_hbm, rbuf, lbuf, ones_b, zbuf, cacc):
        cid = lax.axis_index("c")
        sid = lax.axis_index("s")
        nbase = cid * HALF

        zero16 = jnp.zeros((16,), F32)
        one16 = jnp.ones((16,), F32)
        for r in range(8):
            for j in range(8):
                zbuf[r, pl.ds(j * 16, 16)] = zero16

        def fill(r, carry):
            for j in range(8):
                ones_b[r, pl.ds(j * 16, 16)] = one16
            return carry

        lax.fori_loop(0, K2, fill, 0)
        nz = (ZCH + NT - 1) // NT
        for c0 in range(nz):
            g = c0 * NT + sid

            @pl.when(g < ZCH)
            def _():
                pltpu.sync_copy(zbuf, cacc.at[pl.ds(g * 8, 8)])

        plsc.subcore_barrier()

        def chunk(i, carry):
            off = sid * TE + i * K2
            pltpu.sync_copy(row_hbm.at[pl.ds(off, K2)], rbuf)
            for j in range(K2 // 16):
                v = rbuf[pl.ds(j * 16, 16)]
                lv = v - nbase
                m = (lv >= 0) & (lv < HALF)
                lbuf[pl.ds(j * 16, 16)] = jnp.where(m, lv, HALF)
            pltpu.sync_copy(ones_b, cacc.at[lbuf], add=True)
            return carry

        lax.fori_loop(0, CH, chunk, 0)
        plsc.subcore_barrier()

        nw = (WCH + NT - 1) // NT
        for c2 in range(nw):
            g = c2 * NT + sid

            @pl.when(g < WCH)
            def _():
                r0 = g * 8
                pltpu.sync_copy(cacc.at[pl.ds(r0, 8)],
                                cnt_hbm.at[pl.ds(nbase + r0, 8)])

    return ck(row)


def _pick_chunk(total, cands):
    for k in cands:
        if k <= total and total % k == 0:
            return k
    raise ValueError(f"no chunk size for {total}")


def _sc_scatter(o_stk, row, num_nodes):
    _, E, HH = o_stk.shape
    NT = 16
    TE = E // NT
    K2 = _pick_chunk(TE, (80, 48, 16))
    CH = TE // K2
    assert CH % 2 == 0
    ZR = 40 if num_nodes % 40 == 0 else 8
    ZCH = num_nodes // ZR

    mesh = plsc.VectorSubcoreMesh(core_axis_name="c", subcore_axis_name="s")

    @functools.partial(
        pl.kernel, mesh=mesh,
        out_type=jax.ShapeDtypeStruct((2, num_nodes, HH), F32),
        scratch_types=[
            [pltpu.VMEM((K2,), jnp.int32) for _ in range(2)],
            [pltpu.VMEM((K2, HH), F32) for _ in range(2)],
            pltpu.VMEM((ZR, HH), F32),
            pltpu.VMEM_SHARED((num_nodes, HH), F32),
            [pltpu.SemaphoreType.DMA for _ in range(2)],
            [pltpu.SemaphoreType.DMA for _ in range(2)],
        ],
    )
    def sk(o_hbm, row_hbm, s_hbm, rbuf, dbuf, zbuf, acc, dsem, ssem):
        cid = lax.axis_index("c")
        sid = lax.axis_index("s")

        zero16 = jnp.zeros((16,), F32)

        def zfill(r, carry):
            for j in range(HH // 16):
                zbuf[r, pl.ds(j * 16, 16)] = zero16
            return carry

        lax.fori_loop(0, ZR, zfill, 0)
        nz = (ZCH + NT - 1) // NT
        for c0 in range(nz):
            g = c0 * NT + sid

            @pl.when(g < ZCH)
            def _():
                pltpu.sync_copy(zbuf, acc.at[pl.ds(g * ZR, ZR)])

        plsc.subcore_barrier()

        def start(b, i):
            off = sid * TE + i * K2
            pltpu.sync_copy(row_hbm.at[pl.ds(off, K2)], rbuf[b])
            pltpu.async_copy(o_hbm.at[cid, pl.ds(off, K2)], dbuf[b],
                             dsem[b])

        def finish(b, i):
            off = sid * TE + i * K2
            pltpu.make_async_copy(o_hbm.at[cid, pl.ds(off, K2)], dbuf[b],
                                  dsem[b]).wait()
            pltpu.async_copy(dbuf[b], acc.at[rbuf[b]], ssem[b], add=True)

        def wait_sc(b):
            pltpu.make_async_copy(dbuf[b], acc.at[rbuf[b]], ssem[b]).wait()

        start(0, 0)
        start(1, 1)

        def body(g, carry):
            i0 = g * 2
            finish(0, i0)
            finish(1, i0 + 1)
            wait_sc(0)
            wait_sc(1)
            start(0, i0 + 2)
            start(1, i0 + 3)
            return carry

        lax.fori_loop(0, CH // 2 - 1, body, 0)
        finish(0, CH - 2)
        finish(1, CH - 1)
        wait_sc(0)
        wait_sc(1)

        plsc.subcore_barrier()

        nw = (ZCH + NT - 1) // NT
        for c2 in range(nw):
            g = c2 * NT + sid

            @pl.when(g < ZCH)
            def _():
                r0 = g * ZR
                pltpu.sync_copy(acc.at[pl.ds(r0, ZR)],
                                s_hbm.at[cid, pl.ds(r0, ZR)])

    return sk(o_stk, row)


def _tc_edge(src, dst, ea, w, store_ea, packed):
    E, FC = src.shape
    FE = ea.shape[1]
    H = w["w2T"].shape[0]
    BE = 1280 if E % 1280 == 0 else 640
    grid = (E // BE,)

    if packed:
        wmm = [w["w1sT"][:FC], w["w1sT"][FC:], w["w1dT"][:FC],
               w["w1dT"][FC:], w["w1eT"][:FE], w["w1eT"][FE:],
               w["v1dT"][:FC], w["v1dT"][FC:]]
    else:
        wmm = [w["w1sT"], w["w1dT"], w["w1eT"], w["v1dT"]]
    wrest = [w["b1"], w["g1"], w["be1"], w["w2T"], w["b2"], w["v1eT"],
             w["c1"], w["g2"], w["be2"], w["v2T"], w["c2"]]
    wlist = wmm + wrest

    in_specs = [
        pl.BlockSpec((BE, FC), lambda i: (i, 0)),
        pl.BlockSpec((BE, FC), lambda i: (i, 0)),
        pl.BlockSpec((BE, FE), lambda i: (i, 0)),
    ] + [pl.BlockSpec(a.shape, lambda i: (0, 0)) for a in wlist]

    HH = H // 2
    osp = [pl.BlockSpec((2, BE, HH), lambda i: (0, i, 0))]
    osh = [jax.ShapeDtypeStruct((2, E, HH), F32)]
    if store_ea:
        out_shape = (jax.ShapeDtypeStruct((E, HH), jnp.int32), *osh)
        out_specs = (pl.BlockSpec((BE, HH), lambda i: (i, 0)), *osp)
    else:
        out_shape = osh[0]
        out_specs = osp[0]

    def body(src_ref, dst_ref, ea_ref, *refs):
        ws = refs[:len(wmm)]
        (b1, g1, be1, w2, b2, v1e, c1, g2, be2, v2, c2) = \
            refs[len(wmm):len(wmm) + len(wrest)]
        outs = refs[len(wmm) + len(wrest):]

        def mm(a, b):
            return jnp.dot(a, b[...], preferred_element_type=F32)

        if packed:
            slo, shi = _unpk(src_ref[...])
            dlo, dhi = _unpk(dst_ref[...])
            elo, ehi = _unpk(ea_ref[...])
            h = (mm(slo, ws[0]) + mm(shi, ws[1]) + mm(dlo, ws[2])
                 + mm(dhi, ws[3]) + mm(elo, ws[4]) + mm(ehi, ws[5]))
        else:
            h = (mm(src_ref[...], ws[0]) + mm(dst_ref[...], ws[1])
                 + mm(ea_ref[...], ws[2]))
        h = _ln(jnp.maximum(h + b1[...], 0.0), g1[...], be1[...])
        ea2 = mm(h, w2) + b2[...]
        h2 = mm(ea2, v1e)
        if packed:
            h2 = h2 + mm(dlo, ws[6]) + mm(dhi, ws[7])
        else:
            h2 = h2 + mm(dst_ref[...], ws[3])
        h2 = _ln(jnp.maximum(h2 + c1[...], 0.0), g2[...], be2[...])
        o = mm(h2, v2) + c2[...]
        if store_ea:
            outs[0][...] = _pk(ea2[:, :HH], ea2[:, HH:])
            outs[1][0] = o[:, :HH]
            outs[1][1] = o[:, HH:]
        else:
            outs[0][0] = o[:, :HH]
            outs[0][1] = o[:, HH:]

    return pl.pallas_call(
        body, grid=grid, in_specs=in_specs, out_specs=out_specs,
        out_shape=out_shape,
    )(src, dst, ea, *wlist)


def _tc_node(x, s_stk, cnt, w):
    N, F = x.shape
    HH = s_stk.shape[2]
    T = w["u2T"].shape[1]
    BN = 400
    grid = (N // BN,)

    wlist = [w["u1xT"], w["u1aT"], w["c1"], w["g"], w["be"], w["u2T"],
             w["c2"]]
    in_specs = [
        pl.BlockSpec((BN, F), lambda i: (i, 0)),
        pl.BlockSpec((2, BN, HH), lambda i: (0, i, 0)),
        pl.BlockSpec((BN, 128), lambda i: (i, 0)),
    ] + [pl.BlockSpec(a.shape, lambda i: (0, 0)) for a in wlist]

    def body(x_ref, s_ref, cnt_ref, u1x, u1a, c1, g, be, u2, c2,
             out_ref):
        inv = 1.0 / jnp.maximum(cnt_ref[:, 0:1], 1.0)
        agg = jnp.concatenate([s_ref[0], s_ref[1]], axis=1) * inv
        h = (jnp.dot(x_ref[...], u1x[...], preferred_element_type=F32)
             + jnp.dot(agg, u1a[...], preferred_element_type=F32))
        h = _ln(jnp.maximum(h + c1[...], 0.0), g[...], be[...])
        out_ref[...] = (jnp.dot(h, u2[...], preferred_element_type=F32)
                        + c2[...])

    return pl.pallas_call(
        body, grid=grid, in_specs=in_specs,
        out_specs=pl.BlockSpec((BN, T), lambda i: (i, 0)),
        out_shape=jax.ShapeDtypeStruct((N, T), F32),
    )(x, s_stk, cnt, *wlist)


def _prep_edge(p_edge, p_node1, F, H):
    w1 = p_edge["w1"]
    v1 = p_node1["w1"]
    return {
        "w1sT": w1[:, :F].T, "w1dT": w1[:, F:2 * F].T,
        "w1eT": w1[:, 2 * F:].T,
        "b1": p_edge["b1"][None, :], "g1": p_edge["g"][None, :],
        "be1": p_edge["be"][None, :], "w2T": p_edge["w2"].T,
        "b2": p_edge["b2"][None, :],
        "v1dT": v1[:, :F].T, "v1eT": v1[:, F:].T,
        "c1": p_node1["b1"][None, :], "g2": p_node1["g"][None, :],
        "be2": p_node1["be"][None, :], "v2T": p_node1["w2"].T,
        "c2": p_node1["b2"][None, :],
    }


def _prep_node2(p, F, H):
    u1 = p["w1"]
    return {
        "u1xT": u1[:, :F].T, "u1aT": u1[:, F:].T,
        "c1": p["b1"][None, :],
        "g": p["g"][None, :], "be": p["be"][None, :],
        "u2T": p["w2"].T,
        "c2": p["b2"][None, :],
    }


def _impl(x, edge_idx, edge_attr, params):
    row = edge_idx[0]
    col = edge_idx[1]
    N = x.shape[0]
    cnt = None
    ea = edge_attr
    for lname in ("l1", "l2", "l3"):
        p = params[lname]
        F = x.shape[1]
        H = p["edge"]["w2"].shape[0]
        last = lname == "l3"
        packed = lname != "l1"
        ew = _prep_edge(p["edge"], p["node1"], F, H)
        nw = _prep_node2(p["node2"], F, H)

        xg = _pack_cols(x) if packed else x
        src, dst = _sc_gather(xg, row, col)
        if last:
            o_stk = _tc_edge(src, dst, ea, ew, store_ea=False,
                             packed=packed)
            ea_next = None
        else:
            ea_next, o_stk = _tc_edge(src, dst, ea, ew, store_ea=True,
                                      packed=packed)
        if cnt is None:
            cnt = _sc_counts(row, N)
        s_stk = _sc_scatter(o_stk, row, N)
        x = _tc_node(x, s_stk, cnt, nw)
        ea = ea_next
    return x


kernel = jax.jit(_impl)


# speedup vs baseline: 2.7175x; 1.0422x over previous
import functools

import jax
import jax.numpy as jnp
from jax import lax
from jax.experimental import pallas as pl
from jax.experimental.pallas import tpu as pltpu
from jax.experimental.pallas import tpu_sc as plsc

F32 = jnp.float32
BF16 = jnp.bfloat16


def _pack_cols(x):
    r, c2 = x.shape
    c = c2 // 2
    xb = x.astype(BF16)
    st = jnp.stack([xb[:, :c], xb[:, c:]], axis=-1)
    return jax.lax.bitcast_convert_type(st, jnp.int32)


def _unpk(v):
    lo = lax.bitcast_convert_type(lax.shift_left(v, 16), F32)
    hi = lax.bitcast_convert_type(v & jnp.int32(-65536), F32)
    return lo, hi


def _pk(lo, hi):
    def rtn(f):
        u = lax.bitcast_convert_type(f, jnp.int32)
        return u + jnp.int32(0x7FFF) + (lax.shift_right_logical(u, 16)
                                        & jnp.int32(1))
    lo_w = lax.shift_right_logical(rtn(lo), 16)
    hi_w = rtn(hi) & jnp.int32(-65536)
    return lo_w | hi_w


def _ln(h, g, be):
    mu = jnp.mean(h, axis=-1, keepdims=True)
    d = h - mu
    var = jnp.mean(d * d, axis=-1, keepdims=True)
    return d * lax.rsqrt(var + 1e-5) * g + be


def _sc_gather(x, row, col):
    N, F = x.shape
    E = row.shape[0]
    NW = 32
    EW = E // NW
    K = 200
    CH = EW // K
    assert CH % 2 == 0 and F <= 128

    mesh = plsc.VectorSubcoreMesh(core_axis_name="c", subcore_axis_name="s")
    dt = x.dtype

    @functools.partial(
        pl.kernel,
        mesh=mesh,
        out_type=(jax.ShapeDtypeStruct((E, F), dt),
                  jax.ShapeDtypeStruct((E, F), dt)),
        scratch_types=[
            [pltpu.VMEM((K,), jnp.int32) for _ in range(2)],
            [pltpu.VMEM((K,), jnp.int32) for _ in range(2)],
            [pltpu.VMEM((K, F), dt) for _ in range(2)],
            [pltpu.VMEM((K, F), dt) for _ in range(2)],
            [pltpu.SemaphoreType.DMA for _ in range(2)],
            [pltpu.SemaphoreType.DMA for _ in range(2)],
            [pltpu.SemaphoreType.DMA for _ in range(2)],
            [pltpu.SemaphoreType.DMA for _ in range(2)],
        ],
    )
    def gk(x_hbm, row_hbm, col_hbm, src_hbm, dst_hbm,
           idx_r, idx_c, buf_r, buf_c, sem_r, sem_c, wsem_r, wsem_c):
        wid = lax.axis_index("s") * 2 + lax.axis_index("c")
        base = wid * EW

        def start(b, i):
            off = base + i * K
            pltpu.sync_copy(row_hbm.at[pl.ds(off, K)], idx_r[b])
            pltpu.sync_copy(col_hbm.at[pl.ds(off, K)], idx_c[b])
            pltpu.async_copy(x_hbm.at[idx_r[b]], buf_r[b], sem_r[b])
            pltpu.async_copy(x_hbm.at[idx_c[b]], buf_c[b], sem_c[b])

        def finish(b, i):
            off = base + i * K
            pltpu.make_async_copy(x_hbm.at[idx_r[b]], buf_r[b],
                                  sem_r[b]).wait()
            pltpu.make_async_copy(x_hbm.at[idx_c[b]], buf_c[b],
                                  sem_c[b]).wait()
            pltpu.async_copy(buf_r[b], src_hbm.at[pl.ds(off, K)], wsem_r[b])
            pltpu.async_copy(buf_c[b], dst_hbm.at[pl.ds(off, K)], wsem_c[b])

        def wait_wb(b, i):
            off = base + i * K
            pltpu.make_async_copy(buf_r[b], src_hbm.at[pl.ds(off, K)],
                                  wsem_r[b]).wait()
            pltpu.make_async_copy(buf_c[b], dst_hbm.at[pl.ds(off, K)],
                                  wsem_c[b]).wait()

        start(0, 0)
        start(1, 1)

        def body(g, carry):
            i0 = g * 2
            finish(0, i0)
            finish(1, i0 + 1)
            wait_wb(0, i0)
            wait_wb(1, i0 + 1)
            start(0, i0 + 2)
            start(1, i0 + 3)
            return carry

        lax.fori_loop(0, CH // 2 - 1, body, 0)
        finish(0, CH - 2)
        finish(1, CH - 1)
        wait_wb(0, CH - 2)
        wait_wb(1, CH - 1)

    return gk(x, row, col)


def _sc_counts(row, num_nodes):
    E = row.shape[0]
    NT = 16
    TE = E // NT
    K2 = _pick_chunk(TE, (400, 80, 16))
    CH = TE // K2
    HALF = num_nodes // 2
    ACC = HALF + 8
    ZCH = ACC // 8
    WCH = HALF // 8

    mesh = plsc.VectorSubcoreMesh(core_axis_name="c", subcore_axis_name="s")

    @functools.partial(
        pl.kernel, mesh=mesh,
        out_type=jax.ShapeDtypeStruct((num_nodes, 128), F32),
        scratch_types=[
            pltpu.VMEM((K2,), jnp.int32),
            pltpu.VMEM((K2,), jnp.int32),
            pltpu.VMEM((K2, 128), F32),
            pltpu.VMEM((8, 128), F32),
            pltpu.VMEM_SHARED((ACC, 128), F32),
        ],
    )
    def ck(row_hbm, cnt_hbm, rbuf, lbuf, ones_b, zbuf, cacc):
        cid = lax.axis_index("c")
        sid = lax.axis_index("s")
        nbase = cid * HALF

        zero16 = jnp.zeros((16,), F32)
        one16 = jnp.ones((16,), F32)
        for r in range(8):
            for j in range(8):
                zbuf[r, pl.ds(j * 16, 16)] = zero16

        def fill(r, carry):
            for j in range(8):
                ones_b[r, pl.ds(j * 16, 16)] = one16
            return carry

        lax.fori_loop(0, K2, fill, 0)
        nz = (ZCH + NT - 1) // NT
        for c0 in range(nz):
            g = c0 * NT + sid

            @pl.when(g < ZCH)
            def _():
                pltpu.sync_copy(zbuf, cacc.at[pl.ds(g * 8, 8)])

        plsc.subcore_barrier()

        def chunk(i, carry):
            off = sid * TE + i * K2
            pltpu.sync_copy(row_hbm.at[pl.ds(off, K2)], rbuf)
            for j in range(K2 // 16):
                v = rbuf[pl.ds(j * 16, 16)]
                lv = v - nbase
                m = (lv >= 0) & (lv < HALF)
                lbuf[pl.ds(j * 16, 16)] = jnp.where(m, lv, HALF)
            pltpu.sync_copy(ones_b, cacc.at[lbuf], add=True)
            return carry

        lax.fori_loop(0, CH, chunk, 0)
        plsc.subcore_barrier()

        nw = (WCH + NT - 1) // NT
        for c2 in range(nw):
            g = c2 * NT + sid

            @pl.when(g < WCH)
            def _():
                r0 = g * 8
                pltpu.sync_copy(cacc.at[pl.ds(r0, 8)],
                                cnt_hbm.at[pl.ds(nbase + r0, 8)])

    return ck(row)


def _pick_chunk(total, cands):
    for k in cands:
        if k <= total and total % k == 0:
            return k
    raise ValueError(f"no chunk size for {total}")


def _sc_scatter(o_stk, row, num_nodes):
    _, E, HH = o_stk.shape
    NT = 16
    TE = E // NT
    K2 = _pick_chunk(TE, (80, 48, 16))
    CH = TE // K2
    assert CH % 2 == 0
    ZR = 40 if num_nodes % 40 == 0 else 8
    ZCH = num_nodes // ZR

    mesh = plsc.VectorSubcoreMesh(core_axis_name="c", subcore_axis_name="s")

    @functools.partial(
        pl.kernel, mesh=mesh,
        out_type=jax.ShapeDtypeStruct((2, num_nodes, HH), F32),
        scratch_types=[
            [pltpu.VMEM((K2,), jnp.int32) for _ in range(2)],
            [pltpu.VMEM((K2, HH), F32) for _ in range(2)],
            pltpu.VMEM((ZR, HH), F32),
            pltpu.VMEM_SHARED((num_nodes, HH), F32),
            [pltpu.SemaphoreType.DMA for _ in range(2)],
            [pltpu.SemaphoreType.DMA for _ in range(2)],
        ],
    )
    def sk(o_hbm, row_hbm, s_hbm, rbuf, dbuf, zbuf, acc, dsem, ssem):
        cid = lax.axis_index("c")
        sid = lax.axis_index("s")

        zero16 = jnp.zeros((16,), F32)

        def zfill(r, carry):
            for j in range(HH // 16):
                zbuf[r, pl.ds(j * 16, 16)] = zero16
            return carry

        lax.fori_loop(0, ZR, zfill, 0)
        nz = (ZCH + NT - 1) // NT
        for c0 in range(nz):
            g = c0 * NT + sid

            @pl.when(g < ZCH)
            def _():
                pltpu.sync_copy(zbuf, acc.at[pl.ds(g * ZR, ZR)])

        plsc.subcore_barrier()

        def start(b, i):
            off = sid * TE + i * K2
            pltpu.sync_copy(row_hbm.at[pl.ds(off, K2)], rbuf[b])
            pltpu.async_copy(o_hbm.at[cid, pl.ds(off, K2)], dbuf[b],
                             dsem[b])

        def finish(b, i):
            off = sid * TE + i * K2
            pltpu.make_async_copy(o_hbm.at[cid, pl.ds(off, K2)], dbuf[b],
                                  dsem[b]).wait()
            pltpu.async_copy(dbuf[b], acc.at[rbuf[b]], ssem[b], add=True)

        def wait_sc(b):
            pltpu.make_async_copy(dbuf[b], acc.at[rbuf[b]], ssem[b]).wait()

        start(0, 0)
        start(1, 1)

        def body(g, carry):
            i0 = g * 2
            finish(0, i0)
            finish(1, i0 + 1)
            wait_sc(0)
            wait_sc(1)
            start(0, i0 + 2)
            start(1, i0 + 3)
            return carry

        lax.fori_loop(0, CH // 2 - 1, body, 0)
        finish(0, CH - 2)
        finish(1, CH - 1)
        wait_sc(0)
        wait_sc(1)

        plsc.subcore_barrier()

        nw = (ZCH + NT - 1) // NT
        for c2 in range(nw):
            g = c2 * NT + sid

            @pl.when(g < ZCH)
            def _():
                r0 = g * ZR
                pltpu.sync_copy(acc.at[pl.ds(r0, ZR)],
                                s_hbm.at[cid, pl.ds(r0, ZR)])

    return sk(o_stk, row)


def _tc_edge(src, dst, ea, w, store_ea, packed):
    E, FC = src.shape
    FE = ea.shape[1]
    H = w["w2T"].shape[0]
    BE = 2560 if E % 2560 == 0 else 640
    grid = (E // BE,)

    if packed:
        wmm = [w["w1sT"][:FC], w["w1sT"][FC:], w["w1dT"][:FC],
               w["w1dT"][FC:], w["w1eT"][:FE], w["w1eT"][FE:],
               w["v1dT"][:FC], w["v1dT"][FC:]]
    else:
        wmm = [w["w1sT"], w["w1dT"], w["w1eT"], w["v1dT"]]
    wrest = [w["b1"], w["g1"], w["be1"], w["w2T"], w["b2"], w["v1eT"],
             w["c1"], w["g2"], w["be2"], w["v2T"], w["c2"]]
    wlist = wmm + wrest

    in_specs = [
        pl.BlockSpec((BE, FC), lambda i: (i, 0)),
        pl.BlockSpec((BE, FC), lambda i: (i, 0)),
        pl.BlockSpec((BE, FE), lambda i: (i, 0)),
    ] + [pl.BlockSpec(a.shape, lambda i: (0, 0)) for a in wlist]

    HH = H // 2
    osp = [pl.BlockSpec((2, BE, HH), lambda i: (0, i, 0))]
    osh = [jax.ShapeDtypeStruct((2, E, HH), F32)]
    if store_ea:
        out_shape = (jax.ShapeDtypeStruct((E, HH), jnp.int32), *osh)
        out_specs = (pl.BlockSpec((BE, HH), lambda i: (i, 0)), *osp)
    else:
        out_shape = osh[0]
        out_specs = osp[0]

    def body(src_ref, dst_ref, ea_ref, *refs):
        ws = refs[:len(wmm)]
        (b1, g1, be1, w2, b2, v1e, c1, g2, be2, v2, c2) = \
            refs[len(wmm):len(wmm) + len(wrest)]
        outs = refs[len(wmm) + len(wrest):]

        def mm(a, b):
            return jnp.dot(a, b[...], preferred_element_type=F32)

        if packed:
            slo, shi = _unpk(src_ref[...])
            dlo, dhi = _unpk(dst_ref[...])
            elo, ehi = _unpk(ea_ref[...])
            h = (mm(slo, ws[0]) + mm(shi, ws[1]) + mm(dlo, ws[2])
                 + mm(dhi, ws[3]) + mm(elo, ws[4]) + mm(ehi, ws[5]))
        else:
            h = (mm(src_ref[...], ws[0]) + mm(dst_ref[...], ws[1])
                 + mm(ea_ref[...], ws[2]))
        h = _ln(jnp.maximum(h + b1[...], 0.0), g1[...], be1[...])
        ea2 = mm(h, w2) + b2[...]
        h2 = mm(ea2, v1e)
        if packed:
            h2 = h2 + mm(dlo, ws[6]) + mm(dhi, ws[7])
        else:
            h2 = h2 + mm(dst_ref[...], ws[3])
        h2 = _ln(jnp.maximum(h2 + c1[...], 0.0), g2[...], be2[...])
        o = mm(h2, v2) + c2[...]
        if store_ea:
            outs[0][...] = _pk(ea2[:, :HH], ea2[:, HH:])
            outs[1][0] = o[:, :HH]
            outs[1][1] = o[:, HH:]
        else:
            outs[0][0] = o[:, :HH]
            outs[0][1] = o[:, HH:]

    return pl.pallas_call(
        body, grid=grid, in_specs=in_specs, out_specs=out_specs,
        out_shape=out_shape,
    )(src, dst, ea, *wlist)


def _tc_node(x, s_stk, cnt, w):
    N, F = x.shape
    HH = s_stk.shape[2]
    T = w["u2T"].shape[1]
    BN = 1000 if N % 1000 == 0 else 400
    grid = (N // BN,)

    wlist = [w["u1xT"], w["u1aT"], w["c1"], w["g"], w["be"], w["u2T"],
             w["c2"]]
    in_specs = [
        pl.BlockSpec((BN, F), lambda i: (i, 0)),
        pl.BlockSpec((2, BN, HH), lambda i: (0, i, 0)),
        pl.BlockSpec((BN, 128), lambda i: (i, 0)),
    ] + [pl.BlockSpec(a.shape, lambda i: (0, 0)) for a in wlist]

    def body(x_ref, s_ref, cnt_ref, u1x, u1a, c1, g, be, u2, c2,
             out_ref):
        inv = 1.0 / jnp.maximum(cnt_ref[:, 0:1], 1.0)
        agg = jnp.concatenate([s_ref[0], s_ref[1]], axis=1) * inv
        h = (jnp.dot(x_ref[...], u1x[...], preferred_element_type=F32)
             + jnp.dot(agg, u1a[...], preferred_element_type=F32))
        h = _ln(jnp.maximum(h + c1[...], 0.0), g[...], be[...])
        out_ref[...] = (jnp.dot(h, u2[...], preferred_element_type=F32)
                        + c2[...])

    return pl.pallas_call(
        body, grid=grid, in_specs=in_specs,
        out_specs=pl.BlockSpec((BN, T), lambda i: (i, 0)),
        out_shape=jax.ShapeDtypeStruct((N, T), F32),
    )(x, s_stk, cnt, *wlist)


def _prep_edge(p_edge, p_node1, F, H):
    w1 = p_edge["w1"]
    v1 = p_node1["w1"]
    return {
        "w1sT": w1[:, :F].T, "w1dT": w1[:, F:2 * F].T,
        "w1eT": w1[:, 2 * F:].T,
        "b1": p_edge["b1"][None, :], "g1": p_edge["g"][None, :],
        "be1": p_edge["be"][None, :], "w2T": p_edge["w2"].T,
        "b2": p_edge["b2"][None, :],
        "v1dT": v1[:, :F].T, "v1eT": v1[:, F:].T,
        "c1": p_node1["b1"][None, :], "g2": p_node1["g"][None, :],
        "be2": p_node1["be"][None, :], "v2T": p_node1["w2"].T,
        "c2": p_node1["b2"][None, :],
    }


def _prep_node2(p, F, H):
    u1 = p["w1"]
    return {
        "u1xT": u1[:, :F].T, "u1aT": u1[:, F:].T,
        "c1": p["b1"][None, :],
        "g": p["g"][None, :], "be": p["be"][None, :],
        "u2T": p["w2"].T,
        "c2": p["b2"][None, :],
    }


def _impl(x, edge_idx, edge_attr, params):
    row = edge_idx[0]
    col = edge_idx[1]
    N = x.shape[0]
    cnt = None
    ea = edge_attr
    for lname in ("l1", "l2", "l3"):
        p = params[lname]
        F = x.shape[1]
        H = p["edge"]["w2"].shape[0]
        last = lname == "l3"
        packed = lname != "l1"
        ew = _prep_edge(p["edge"], p["node1"], F, H)
        nw = _prep_node2(p["node2"], F, H)

        xg = _pack_cols(x) if packed else x
        src, dst = _sc_gather(xg, row, col)
        if last:
            o_stk = _tc_edge(src, dst, ea, ew, store_ea=False,
                             packed=packed)
            ea_next = None
        else:
            ea_next, o_stk = _tc_edge(src, dst, ea, ew, store_ea=True,
                                      packed=packed)
        if cnt is None:
            cnt = _sc_counts(row, N)
        s_stk = _sc_scatter(o_stk, row, N)
        x = _tc_node(x, s_stk, cnt, nw)
        ea = ea_next
    return x


kernel = jax.jit(_impl)


# speedup vs baseline: 2.7479x; 1.0112x over previous
import functools

import jax
import jax.numpy as jnp
from jax import lax
from jax.experimental import pallas as pl
from jax.experimental.pallas import tpu as pltpu
from jax.experimental.pallas import tpu_sc as plsc

F32 = jnp.float32
BF16 = jnp.bfloat16


def _pack_cols(x):
    r, c2 = x.shape
    c = c2 // 2
    xb = x.astype(BF16)
    st = jnp.stack([xb[:, :c], xb[:, c:]], axis=-1)
    return jax.lax.bitcast_convert_type(st, jnp.int32)


def _unpk(v):
    lo = lax.bitcast_convert_type(lax.shift_left(v, 16), F32)
    hi = lax.bitcast_convert_type(v & jnp.int32(-65536), F32)
    return lo, hi


def _pk(lo, hi):
    def rtn(f):
        u = lax.bitcast_convert_type(f, jnp.int32)
        return u + jnp.int32(0x7FFF) + (lax.shift_right_logical(u, 16)
                                        & jnp.int32(1))
    lo_w = lax.shift_right_logical(rtn(lo), 16)
    hi_w = rtn(hi) & jnp.int32(-65536)
    return lo_w | hi_w


def _ln(h, g, be):
    mu = jnp.mean(h, axis=-1, keepdims=True)
    d = h - mu
    var = jnp.mean(d * d, axis=-1, keepdims=True)
    return d * lax.rsqrt(var + 1e-5) * g + be


def _sc_gather(x, row, col):
    N, F = x.shape
    E = row.shape[0]
    NW = 32
    EW = E // NW
    K = 200
    CH = EW // K
    assert CH % 2 == 0 and F <= 128

    mesh = plsc.VectorSubcoreMesh(core_axis_name="c", subcore_axis_name="s")
    dt = x.dtype

    @functools.partial(
        pl.kernel,
        mesh=mesh,
        out_type=(jax.ShapeDtypeStruct((E, F), dt),
                  jax.ShapeDtypeStruct((E, F), dt)),
        scratch_types=[
            [pltpu.VMEM((K,), jnp.int32) for _ in range(2)],
            [pltpu.VMEM((K,), jnp.int32) for _ in range(2)],
            [pltpu.VMEM((K, F), dt) for _ in range(2)],
            [pltpu.VMEM((K, F), dt) for _ in range(2)],
            [pltpu.SemaphoreType.DMA for _ in range(2)],
            [pltpu.SemaphoreType.DMA for _ in range(2)],
            [pltpu.SemaphoreType.DMA for _ in range(2)],
            [pltpu.SemaphoreType.DMA for _ in range(2)],
        ],
    )
    def gk(x_hbm, row_hbm, col_hbm, src_hbm, dst_hbm,
           idx_r, idx_c, buf_r, buf_c, sem_r, sem_c, wsem_r, wsem_c):
        wid = lax.axis_index("s") * 2 + lax.axis_index("c")
        base = wid * EW

        def start(b, i):
            off = base + i * K
            pltpu.sync_copy(row_hbm.at[pl.ds(off, K)], idx_r[b])
            pltpu.sync_copy(col_hbm.at[pl.ds(off, K)], idx_c[b])
            pltpu.async_copy(x_hbm.at[idx_r[b]], buf_r[b], sem_r[b])
            pltpu.async_copy(x_hbm.at[idx_c[b]], buf_c[b], sem_c[b])

        def finish(b, i):
            off = base + i * K
            pltpu.make_async_copy(x_hbm.at[idx_r[b]], buf_r[b],
                                  sem_r[b]).wait()
            pltpu.make_async_copy(x_hbm.at[idx_c[b]], buf_c[b],
                                  sem_c[b]).wait()
            pltpu.async_copy(buf_r[b], src_hbm.at[pl.ds(off, K)], wsem_r[b])
            pltpu.async_copy(buf_c[b], dst_hbm.at[pl.ds(off, K)], wsem_c[b])

        def wait_wb(b, i):
            off = base + i * K
            pltpu.make_async_copy(buf_r[b], src_hbm.at[pl.ds(off, K)],
                                  wsem_r[b]).wait()
            pltpu.make_async_copy(buf_c[b], dst_hbm.at[pl.ds(off, K)],
                                  wsem_c[b]).wait()

        start(0, 0)
        start(1, 1)

        def body(g, carry):
            i0 = g * 2
            finish(0, i0)
            finish(1, i0 + 1)
            wait_wb(0, i0)
            wait_wb(1, i0 + 1)
            start(0, i0 + 2)
            start(1, i0 + 3)
            return carry

        lax.fori_loop(0, CH // 2 - 1, body, 0)
        finish(0, CH - 2)
        finish(1, CH - 1)
        wait_wb(0, CH - 2)
        wait_wb(1, CH - 1)

    return gk(x, row, col)


def _sc_counts(row, num_nodes):
    E = row.shape[0]
    NT = 16
    TE = E // NT
    K2 = _pick_chunk(TE, (400, 80, 16))
    CH = TE // K2
    HALF = num_nodes // 2
    ACC = HALF + 8
    ZCH = ACC // 8
    WCH = HALF // 8

    mesh = plsc.VectorSubcoreMesh(core_axis_name="c", subcore_axis_name="s")

    @functools.partial(
        pl.kernel, mesh=mesh,
        out_type=jax.ShapeDtypeStruct((num_nodes, 128), F32),
        scratch_types=[
            pltpu.VMEM((K2,), jnp.int32),
            pltpu.VMEM((K2,), jnp.int32),
            pltpu.VMEM((K2, 128), F32),
            pltpu.VMEM((8, 128), F32),
            pltpu.VMEM_SHARED((ACC, 128), F32),
        ],
    )
    def ck(row_hbm, cnt_hbm, rbuf, lbuf, ones_b, zbuf, cacc):
        cid = lax.axis_index("c")
        sid = lax.axis_index("s")
        nbase = cid * HALF

        zero16 = jnp.zeros((16,), F32)
        one16 = jnp.ones((16,), F32)
        for r in range(8):
            for j in range(8):
                zbuf[r, pl.ds(j * 16, 16)] = zero16

        def fill(r, carry):
            for j in range(8):
                ones_b[r, pl.ds(j * 16, 16)] = one16
            return carry

        lax.fori_loop(0, K2, fill, 0)
        nz = (ZCH + NT - 1) // NT
        for c0 in range(nz):
            g = c0 * NT + sid

            @pl.when(g < ZCH)
            def _():
                pltpu.sync_copy(zbuf, cacc.at[pl.ds(g * 8, 8)])

        plsc.subcore_barrier()

        def chunk(i, carry):
            off = sid * TE + i * K2
            pltpu.sync_copy(row_hbm.at[pl.ds(off, K2)], rbuf)
            for j in range(K2 // 16):
                v = rbuf[pl.ds(j * 16, 16)]
                lv = v - nbase
                m = (lv >= 0) & (lv < HALF)
                lbuf[pl.ds(j * 16, 16)] = jnp.where(m, lv, HALF)
            pltpu.sync_copy(ones_b, cacc.at[lbuf], add=True)
            return carry

        lax.fori_loop(0, CH, chunk, 0)
        plsc.subcore_barrier()

        nw = (WCH + NT - 1) // NT
        for c2 in range(nw):
            g = c2 * NT + sid

            @pl.when(g < WCH)
            def _():
                r0 = g * 8
                pltpu.sync_copy(cacc.at[pl.ds(r0, 8)],
                                cnt_hbm.at[pl.ds(nbase + r0, 8)])

    return ck(row)


def _pick_chunk(total, cands):
    for k in cands:
        if k <= total and total % k == 0:
            return k
    raise ValueError(f"no chunk size for {total}")


def _sc_scatter(o_stk, row, num_nodes):
    _, E, HH = o_stk.shape
    NT = 16
    TE = E // NT
    K2 = _pick_chunk(TE, (80, 48, 16))
    CH = TE // K2
    assert CH % 2 == 0
    ZR = 40 if num_nodes % 40 == 0 else 8
    ZCH = num_nodes // ZR

    mesh = plsc.VectorSubcoreMesh(core_axis_name="c", subcore_axis_name="s")

    @functools.partial(
        pl.kernel, mesh=mesh,
        out_type=jax.ShapeDtypeStruct((2, num_nodes, HH), F32),
        scratch_types=[
            [pltpu.VMEM((K2,), jnp.int32) for _ in range(2)],
            [pltpu.VMEM((K2, HH), F32) for _ in range(2)],
            pltpu.VMEM((ZR, HH), F32),
            pltpu.VMEM_SHARED((num_nodes, HH), F32),
            [pltpu.SemaphoreType.DMA for _ in range(2)],
            [pltpu.SemaphoreType.DMA for _ in range(2)],
        ],
    )
    def sk(o_hbm, row_hbm, s_hbm, rbuf, dbuf, zbuf, acc, dsem, ssem):
        cid = lax.axis_index("c")
        sid = lax.axis_index("s")

        zero16 = jnp.zeros((16,), F32)

        def zfill(r, carry):
            for j in range(HH // 16):
                zbuf[r, pl.ds(j * 16, 16)] = zero16
            return carry

        lax.fori_loop(0, ZR, zfill, 0)
        nz = (ZCH + NT - 1) // NT
        for c0 in range(nz):
            g = c0 * NT + sid

            @pl.when(g < ZCH)
            def _():
                pltpu.sync_copy(zbuf, acc.at[pl.ds(g * ZR, ZR)])

        plsc.subcore_barrier()

        def start(b, i):
            off = sid * TE + i * K2
            pltpu.sync_copy(row_hbm.at[pl.ds(off, K2)], rbuf[b])
            pltpu.async_copy(o_hbm.at[cid, pl.ds(off, K2)], dbuf[b],
                             dsem[b])

        def finish(b, i):
            off = sid * TE + i * K2
            pltpu.make_async_copy(o_hbm.at[cid, pl.ds(off, K2)], dbuf[b],
                                  dsem[b]).wait()
            pltpu.async_copy(dbuf[b], acc.at[rbuf[b]], ssem[b], add=True)

        def wait_sc(b):
            pltpu.make_async_copy(dbuf[b], acc.at[rbuf[b]], ssem[b]).wait()

        start(0, 0)
        start(1, 1)

        def body(g, carry):
            i0 = g * 2
            finish(0, i0)
            finish(1, i0 + 1)
            wait_sc(0)
            wait_sc(1)
            start(0, i0 + 2)
            start(1, i0 + 3)
            return carry

        lax.fori_loop(0, CH // 2 - 1, body, 0)
        finish(0, CH - 2)
        finish(1, CH - 1)
        wait_sc(0)
        wait_sc(1)

        plsc.subcore_barrier()

        nw = (ZCH + NT - 1) // NT
        for c2 in range(nw):
            g = c2 * NT + sid

            @pl.when(g < ZCH)
            def _():
                r0 = g * ZR
                pltpu.sync_copy(acc.at[pl.ds(r0, ZR)],
                                s_hbm.at[cid, pl.ds(r0, ZR)])

    return sk(o_stk, row)


def _tc_edge(src, dst, ea, w, store_ea, packed):
    E, FC = src.shape
    FE = ea.shape[1]
    H = w["w2T"].shape[0]
    BE = 4000 if E % 4000 == 0 else 640
    grid = (E // BE,)

    if packed:
        wmm = [w["w1sT"][:FC], w["w1sT"][FC:], w["w1dT"][:FC],
               w["w1dT"][FC:], w["w1eT"][:FE], w["w1eT"][FE:],
               w["v1dT"][:FC], w["v1dT"][FC:]]
    else:
        wmm = [w["w1sT"], w["w1dT"], w["w1eT"], w["v1dT"]]
    wrest = [w["b1"], w["g1"], w["be1"], w["w2T"], w["b2"], w["v1eT"],
             w["c1"], w["g2"], w["be2"], w["v2T"], w["c2"]]
    wlist = wmm + wrest

    in_specs = [
        pl.BlockSpec((BE, FC), lambda i: (i, 0)),
        pl.BlockSpec((BE, FC), lambda i: (i, 0)),
        pl.BlockSpec((BE, FE), lambda i: (i, 0)),
    ] + [pl.BlockSpec(a.shape, lambda i: (0, 0)) for a in wlist]

    HH = H // 2
    osp = [pl.BlockSpec((2, BE, HH), lambda i: (0, i, 0))]
    osh = [jax.ShapeDtypeStruct((2, E, HH), F32)]
    if store_ea:
        out_shape = (jax.ShapeDtypeStruct((E, HH), jnp.int32), *osh)
        out_specs = (pl.BlockSpec((BE, HH), lambda i: (i, 0)), *osp)
    else:
        out_shape = osh[0]
        out_specs = osp[0]

    def body(src_ref, dst_ref, ea_ref, *refs):
        ws = refs[:len(wmm)]
        (b1, g1, be1, w2, b2, v1e, c1, g2, be2, v2, c2) = \
            refs[len(wmm):len(wmm) + len(wrest)]
        outs = refs[len(wmm) + len(wrest):]

        def mm(a, b):
            return jnp.dot(a, b[...], preferred_element_type=F32)

        if packed:
            slo, shi = _unpk(src_ref[...])
            dlo, dhi = _unpk(dst_ref[...])
            elo, ehi = _unpk(ea_ref[...])
            h = (mm(slo, ws[0]) + mm(shi, ws[1]) + mm(dlo, ws[2])
                 + mm(dhi, ws[3]) + mm(elo, ws[4]) + mm(ehi, ws[5]))
        else:
            h = (mm(src_ref[...], ws[0]) + mm(dst_ref[...], ws[1])
                 + mm(ea_ref[...], ws[2]))
        h = _ln(jnp.maximum(h + b1[...], 0.0), g1[...], be1[...])
        ea2 = mm(h, w2) + b2[...]
        h2 = mm(ea2, v1e)
        if packed:
            h2 = h2 + mm(dlo, ws[6]) + mm(dhi, ws[7])
        else:
            h2 = h2 + mm(dst_ref[...], ws[3])
        h2 = _ln(jnp.maximum(h2 + c1[...], 0.0), g2[...], be2[...])
        o = mm(h2, v2) + c2[...]
        if store_ea:
            outs[0][...] = _pk(ea2[:, :HH], ea2[:, HH:])
            outs[1][0] = o[:, :HH]
            outs[1][1] = o[:, HH:]
        else:
            outs[0][0] = o[:, :HH]
            outs[0][1] = o[:, HH:]

    return pl.pallas_call(
        body, grid=grid, in_specs=in_specs, out_specs=out_specs,
        out_shape=out_shape,
    )(src, dst, ea, *wlist)


def _tc_node(x, s_stk, cnt, w):
    N, F = x.shape
    HH = s_stk.shape[2]
    T = w["u2T"].shape[1]
    BN = 1000 if N % 1000 == 0 else 400
    grid = (N // BN,)

    wlist = [w["u1xT"], w["u1aT"], w["c1"], w["g"], w["be"], w["u2T"],
             w["c2"]]
    in_specs = [
        pl.BlockSpec((BN, F), lambda i: (i, 0)),
        pl.BlockSpec((2, BN, HH), lambda i: (0, i, 0)),
        pl.BlockSpec((BN, 128), lambda i: (i, 0)),
    ] + [pl.BlockSpec(a.shape, lambda i: (0, 0)) for a in wlist]

    def body(x_ref, s_ref, cnt_ref, u1x, u1a, c1, g, be, u2, c2,
             out_ref):
        inv = 1.0 / jnp.maximum(cnt_ref[:, 0:1], 1.0)
        agg = jnp.concatenate([s_ref[0], s_ref[1]], axis=1) * inv
        h = (jnp.dot(x_ref[...], u1x[...], preferred_element_type=F32)
             + jnp.dot(agg, u1a[...], preferred_element_type=F32))
        h = _ln(jnp.maximum(h + c1[...], 0.0), g[...], be[...])
        out_ref[...] = (jnp.dot(h, u2[...], preferred_element_type=F32)
                        + c2[...])

    return pl.pallas_call(
        body, grid=grid, in_specs=in_specs,
        out_specs=pl.BlockSpec((BN, T), lambda i: (i, 0)),
        out_shape=jax.ShapeDtypeStruct((N, T), F32),
    )(x, s_stk, cnt, *wlist)


def _prep_edge(p_edge, p_node1, F, H):
    w1 = p_edge["w1"]
    v1 = p_node1["w1"]
    return {
        "w1sT": w1[:, :F].T, "w1dT": w1[:, F:2 * F].T,
        "w1eT": w1[:, 2 * F:].T,
        "b1": p_edge["b1"][None, :], "g1": p_edge["g"][None, :],
        "be1": p_edge["be"][None, :], "w2T": p_edge["w2"].T,
        "b2": p_edge["b2"][None, :],
        "v1dT": v1[:, :F].T, "v1eT": v1[:, F:].T,
        "c1": p_node1["b1"][None, :], "g2": p_node1["g"][None, :],
        "be2": p_node1["be"][None, :], "v2T": p_node1["w2"].T,
        "c2": p_node1["b2"][None, :],
    }


def _prep_node2(p, F, H):
    u1 = p["w1"]
    return {
        "u1xT": u1[:, :F].T, "u1aT": u1[:, F:].T,
        "c1": p["b1"][None, :],
        "g": p["g"][None, :], "be": p["be"][None, :],
        "u2T": p["w2"].T,
        "c2": p["b2"][None, :],
    }


def _impl(x, edge_idx, edge_attr, params):
    row = edge_idx[0]
    col = edge_idx[1]
    N = x.shape[0]
    cnt = None
    ea = edge_attr
    for lname in ("l1", "l2", "l3"):
        p = params[lname]
        F = x.shape[1]
        H = p["edge"]["w2"].shape[0]
        last = lname == "l3"
        packed = lname != "l1"
        ew = _prep_edge(p["edge"], p["node1"], F, H)
        nw = _prep_node2(p["node2"], F, H)

        xg = _pack_cols(x) if packed else x
        src, dst = _sc_gather(xg, row, col)
        if last:
            o_stk = _tc_edge(src, dst, ea, ew, store_ea=False,
                             packed=packed)
            ea_next = None
        else:
            ea_next, o_stk = _tc_edge(src, dst, ea, ew, store_ea=True,
                                      packed=packed)
        if cnt is None:
            cnt = _sc_counts(row, N)
        s_stk = _sc_scatter(o_stk, row, N)
        x = _tc_node(x, s_stk, cnt, nw)
        ea = ea_next
    return x


kernel = jax.jit(_impl)


# speedup vs baseline: 2.7690x; 1.0077x over previous
import functools

import jax
import jax.numpy as jnp
from jax import lax
from jax.experimental import pallas as pl
from jax.experimental.pallas import tpu as pltpu
from jax.experimental.pallas import tpu_sc as plsc

F32 = jnp.float32
BF16 = jnp.bfloat16


def _pack_cols(x):
    r, c2 = x.shape
    c = c2 // 2
    xb = x.astype(BF16)
    st = jnp.stack([xb[:, :c], xb[:, c:]], axis=-1)
    return jax.lax.bitcast_convert_type(st, jnp.int32)


def _unpk(v):
    lo = lax.bitcast_convert_type(lax.shift_left(v, 16), F32)
    hi = lax.bitcast_convert_type(v & jnp.int32(-65536), F32)
    return lo, hi


def _pk(lo, hi):
    def rtn(f):
        u = lax.bitcast_convert_type(f, jnp.int32)
        return u + jnp.int32(0x7FFF) + (lax.shift_right_logical(u, 16)
                                        & jnp.int32(1))
    lo_w = lax.shift_right_logical(rtn(lo), 16)
    hi_w = rtn(hi) & jnp.int32(-65536)
    return lo_w | hi_w


def _ln(h, g, be):
    mu = jnp.mean(h, axis=-1, keepdims=True)
    d = h - mu
    var = jnp.mean(d * d, axis=-1, keepdims=True)
    return d * lax.rsqrt(var + 1e-5) * g + be


def _sc_gather(x, row, col):
    N, F = x.shape
    E = row.shape[0]
    NW = 32
    EW = E // NW
    K = 200
    CH = EW // K
    assert CH % 2 == 0 and F <= 128

    mesh = plsc.VectorSubcoreMesh(core_axis_name="c", subcore_axis_name="s")
    dt = x.dtype

    @functools.partial(
        pl.kernel,
        mesh=mesh,
        out_type=(jax.ShapeDtypeStruct((E, F), dt),
                  jax.ShapeDtypeStruct((E, F), dt)),
        scratch_types=[
            [pltpu.VMEM((K,), jnp.int32) for _ in range(2)],
            [pltpu.VMEM((K,), jnp.int32) for _ in range(2)],
            [pltpu.VMEM((K, F), dt) for _ in range(2)],
            [pltpu.VMEM((K, F), dt) for _ in range(2)],
            [pltpu.SemaphoreType.DMA for _ in range(2)],
            [pltpu.SemaphoreType.DMA for _ in range(2)],
            [pltpu.SemaphoreType.DMA for _ in range(2)],
            [pltpu.SemaphoreType.DMA for _ in range(2)],
        ],
    )
    def gk(x_hbm, row_hbm, col_hbm, src_hbm, dst_hbm,
           idx_r, idx_c, buf_r, buf_c, sem_r, sem_c, wsem_r, wsem_c):
        wid = lax.axis_index("s") * 2 + lax.axis_index("c")
        base = wid * EW

        def start(b, i):
            off = base + i * K
            pltpu.sync_copy(row_hbm.at[pl.ds(off, K)], idx_r[b])
            pltpu.sync_copy(col_hbm.at[pl.ds(off, K)], idx_c[b])
            pltpu.async_copy(x_hbm.at[idx_r[b]], buf_r[b], sem_r[b])
            pltpu.async_copy(x_hbm.at[idx_c[b]], buf_c[b], sem_c[b])

        def finish(b, i):
            off = base + i * K
            pltpu.make_async_copy(x_hbm.at[idx_r[b]], buf_r[b],
                                  sem_r[b]).wait()
            pltpu.make_async_copy(x_hbm.at[idx_c[b]], buf_c[b],
                                  sem_c[b]).wait()
            pltpu.async_copy(buf_r[b], src_hbm.at[pl.ds(off, K)], wsem_r[b])
            pltpu.async_copy(buf_c[b], dst_hbm.at[pl.ds(off, K)], wsem_c[b])

        def wait_wb(b, i):
            off = base + i * K
            pltpu.make_async_copy(buf_r[b], src_hbm.at[pl.ds(off, K)],
                                  wsem_r[b]).wait()
            pltpu.make_async_copy(buf_c[b], dst_hbm.at[pl.ds(off, K)],
                                  wsem_c[b]).wait()

        start(0, 0)
        start(1, 1)

        def body(g, carry):
            i0 = g * 2
            finish(0, i0)
            finish(1, i0 + 1)
            wait_wb(0, i0)
            wait_wb(1, i0 + 1)
            start(0, i0 + 2)
            start(1, i0 + 3)
            return carry

        lax.fori_loop(0, CH // 2 - 1, body, 0)
        finish(0, CH - 2)
        finish(1, CH - 1)
        wait_wb(0, CH - 2)
        wait_wb(1, CH - 1)

    return gk(x, row, col)


def _sc_counts(row, num_nodes):
    E = row.shape[0]
    NT = 16
    TE = E // NT
    K2 = _pick_chunk(TE, (400, 80, 16))
    CH = TE // K2
    HALF = num_nodes // 2
    ACC = HALF + 8
    ZCH = ACC // 8
    WCH = HALF // 8

    mesh = plsc.VectorSubcoreMesh(core_axis_name="c", subcore_axis_name="s")

    @functools.partial(
        pl.kernel, mesh=mesh,
        out_type=jax.ShapeDtypeStruct((num_nodes, 128), F32),
        scratch_types=[
            pltpu.VMEM((K2,), jnp.int32),
            pltpu.VMEM((K2,), jnp.int32),
            pltpu.VMEM((K2, 128), F32),
            pltpu.VMEM((8, 128), F32),
            pltpu.VMEM_SHARED((ACC, 128), F32),
        ],
    )
    def ck(row_hbm, cnt_hbm, rbuf, lbuf, ones_b, zbuf, cacc):
        cid = lax.axis_index("c")
        sid = lax.axis_index("s")
        nbase = cid * HALF

        zero16 = jnp.zeros((16,), F32)
        one16 = jnp.ones((16,), F32)
        for r in range(8):
            for j in range(8):
                zbuf[r, pl.ds(j * 16, 16)] = zero16

        def fill(r, carry):
            for j in range(8):
                ones_b[r, pl.ds(j * 16, 16)] = one16
            return carry

        lax.fori_loop(0, K2, fill, 0)
        nz = (ZCH + NT - 1) // NT
        for c0 in range(nz):
            g = c0 * NT + sid

            @pl.when(g < ZCH)
            def _():
                pltpu.sync_copy(zbuf, cacc.at[pl.ds(g * 8, 8)])

        plsc.subcore_barrier()

        def chunk(i, carry):
            off = sid * TE + i * K2
            pltpu.sync_copy(row_hbm.at[pl.ds(off, K2)], rbuf)
            for j in range(K2 // 16):
                v = rbuf[pl.ds(j * 16, 16)]
                lv = v - nbase
                m = (lv >= 0) & (lv < HALF)
                lbuf[pl.ds(j * 16, 16)] = jnp.where(m, lv, HALF)
            pltpu.sync_copy(ones_b, cacc.at[lbuf], add=True)
            return carry

        lax.fori_loop(0, CH, chunk, 0)
        plsc.subcore_barrier()

        nw = (WCH + NT - 1) // NT
        for c2 in range(nw):
            g = c2 * NT + sid

            @pl.when(g < WCH)
            def _():
                r0 = g * 8
                pltpu.sync_copy(cacc.at[pl.ds(r0, 8)],
                                cnt_hbm.at[pl.ds(nbase + r0, 8)])

    return ck(row)


def _pick_chunk(total, cands):
    for k in cands:
        if k <= total and total % k == 0:
            return k
    raise ValueError(f"no chunk size for {total}")


def _sc_scatter(o_stk, row, num_nodes):
    _, E, HH = o_stk.shape
    NT = 16
    TE = E // NT
    K2 = _pick_chunk(TE, (80, 48, 16))
    CH = TE // K2
    assert CH % 2 == 0
    ZR = 40 if num_nodes % 40 == 0 else 8
    ZCH = num_nodes // ZR

    mesh = plsc.VectorSubcoreMesh(core_axis_name="c", subcore_axis_name="s")

    @functools.partial(
        pl.kernel, mesh=mesh,
        out_type=jax.ShapeDtypeStruct((2, num_nodes, HH), F32),
        scratch_types=[
            [pltpu.VMEM((K2,), jnp.int32) for _ in range(2)],
            [pltpu.VMEM((K2, HH), F32) for _ in range(2)],
            pltpu.VMEM((ZR, HH), F32),
            pltpu.VMEM_SHARED((num_nodes, HH), F32),
            [pltpu.SemaphoreType.DMA for _ in range(2)],
            [pltpu.SemaphoreType.DMA for _ in range(2)],
        ],
    )
    def sk(o_hbm, row_hbm, s_hbm, rbuf, dbuf, zbuf, acc, dsem, ssem):
        cid = lax.axis_index("c")
        sid = lax.axis_index("s")

        zero16 = jnp.zeros((16,), F32)

        def zfill(r, carry):
            for j in range(HH // 16):
                zbuf[r, pl.ds(j * 16, 16)] = zero16
            return carry

        lax.fori_loop(0, ZR, zfill, 0)
        nz = (ZCH + NT - 1) // NT
        for c0 in range(nz):
            g = c0 * NT + sid

            @pl.when(g < ZCH)
            def _():
                pltpu.sync_copy(zbuf, acc.at[pl.ds(g * ZR, ZR)])

        plsc.subcore_barrier()

        def start(b, i):
            off = sid * TE + i * K2
            pltpu.sync_copy(row_hbm.at[pl.ds(off, K2)], rbuf[b])
            pltpu.async_copy(o_hbm.at[cid, pl.ds(off, K2)], dbuf[b],
                             dsem[b])

        def finish(b, i):
            off = sid * TE + i * K2
            pltpu.make_async_copy(o_hbm.at[cid, pl.ds(off, K2)], dbuf[b],
                                  dsem[b]).wait()
            pltpu.async_copy(dbuf[b], acc.at[rbuf[b]], ssem[b], add=True)

        def wait_sc(b):
            pltpu.make_async_copy(dbuf[b], acc.at[rbuf[b]], ssem[b]).wait()

        start(0, 0)
        start(1, 1)

        def body(g, carry):
            i0 = g * 2
            finish(0, i0)
            finish(1, i0 + 1)
            wait_sc(0)
            wait_sc(1)
            start(0, i0 + 2)
            start(1, i0 + 3)
            return carry

        lax.fori_loop(0, CH // 2 - 1, body, 0)
        finish(0, CH - 2)
        finish(1, CH - 1)
        wait_sc(0)
        wait_sc(1)

        plsc.subcore_barrier()

        nw = (ZCH + NT - 1) // NT
        for c2 in range(nw):
            g = c2 * NT + sid

            @pl.when(g < ZCH)
            def _():
                r0 = g * ZR
                pltpu.sync_copy(acc.at[pl.ds(r0, ZR)],
                                s_hbm.at[cid, pl.ds(r0, ZR)])

    return sk(o_stk, row)


def _tc_edge(src, dst, ea, w, store_ea, packed):
    E, FC = src.shape
    FE = ea.shape[1]
    H = w["w2T"].shape[0]
    BE = 6400 if E % 6400 == 0 else 640
    grid = (E // BE,)

    if packed:
        wmm = [w["w1sT"][:FC], w["w1sT"][FC:], w["w1dT"][:FC],
               w["w1dT"][FC:], w["w1eT"][:FE], w["w1eT"][FE:],
               w["v1dT"][:FC], w["v1dT"][FC:]]
    else:
        wmm = [w["w1sT"], w["w1dT"], w["w1eT"], w["v1dT"]]
    wrest = [w["b1"], w["g1"], w["be1"], w["w2T"], w["b2"], w["v1eT"],
             w["c1"], w["g2"], w["be2"], w["v2T"], w["c2"]]
    wlist = wmm + wrest

    in_specs = [
        pl.BlockSpec((BE, FC), lambda i: (i, 0)),
        pl.BlockSpec((BE, FC), lambda i: (i, 0)),
        pl.BlockSpec((BE, FE), lambda i: (i, 0)),
    ] + [pl.BlockSpec(a.shape, lambda i: (0, 0)) for a in wlist]

    HH = H // 2
    osp = [pl.BlockSpec((2, BE, HH), lambda i: (0, i, 0))]
    osh = [jax.ShapeDtypeStruct((2, E, HH), F32)]
    if store_ea:
        out_shape = (jax.ShapeDtypeStruct((E, HH), jnp.int32), *osh)
        out_specs = (pl.BlockSpec((BE, HH), lambda i: (i, 0)), *osp)
    else:
        out_shape = osh[0]
        out_specs = osp[0]

    def body(src_ref, dst_ref, ea_ref, *refs):
        ws = refs[:len(wmm)]
        (b1, g1, be1, w2, b2, v1e, c1, g2, be2, v2, c2) = \
            refs[len(wmm):len(wmm) + len(wrest)]
        outs = refs[len(wmm) + len(wrest):]

        def mm(a, b):
            return jnp.dot(a, b[...], preferred_element_type=F32)

        if packed:
            slo, shi = _unpk(src_ref[...])
            dlo, dhi = _unpk(dst_ref[...])
            elo, ehi = _unpk(ea_ref[...])
            h = (mm(slo, ws[0]) + mm(shi, ws[1]) + mm(dlo, ws[2])
                 + mm(dhi, ws[3]) + mm(elo, ws[4]) + mm(ehi, ws[5]))
        else:
            h = (mm(src_ref[...], ws[0]) + mm(dst_ref[...], ws[1])
                 + mm(ea_ref[...], ws[2]))
        h = _ln(jnp.maximum(h + b1[...], 0.0), g1[...], be1[...])
        ea2 = mm(h, w2) + b2[...]
        h2 = mm(ea2, v1e)
        if packed:
            h2 = h2 + mm(dlo, ws[6]) + mm(dhi, ws[7])
        else:
            h2 = h2 + mm(dst_ref[...], ws[3])
        h2 = _ln(jnp.maximum(h2 + c1[...], 0.0), g2[...], be2[...])
        o = mm(h2, v2) + c2[...]
        if store_ea:
            outs[0][...] = _pk(ea2[:, :HH], ea2[:, HH:])
            outs[1][0] = o[:, :HH]
            outs[1][1] = o[:, HH:]
        else:
            outs[0][0] = o[:, :HH]
            outs[0][1] = o[:, HH:]

    return pl.pallas_call(
        body, grid=grid, in_specs=in_specs, out_specs=out_specs,
        out_shape=out_shape,
    )(src, dst, ea, *wlist)


def _tc_node(x, s_stk, cnt, w):
    N, F = x.shape
    HH = s_stk.shape[2]
    T = w["u2T"].shape[1]
    BN = 1000 if N % 1000 == 0 else 400
    grid = (N // BN,)

    wlist = [w["u1xT"], w["u1aT"], w["c1"], w["g"], w["be"], w["u2T"],
             w["c2"]]
    in_specs = [
        pl.BlockSpec((BN, F), lambda i: (i, 0)),
        pl.BlockSpec((2, BN, HH), lambda i: (0, i, 0)),
        pl.BlockSpec((BN, 128), lambda i: (i, 0)),
    ] + [pl.BlockSpec(a.shape, lambda i: (0, 0)) for a in wlist]

    def body(x_ref, s_ref, cnt_ref, u1x, u1a, c1, g, be, u2, c2,
             out_ref):
        inv = 1.0 / jnp.maximum(cnt_ref[:, 0:1], 1.0)
        agg = jnp.concatenate([s_ref[0], s_ref[1]], axis=1) * inv
        h = (jnp.dot(x_ref[...], u1x[...], preferred_element_type=F32)
             + jnp.dot(agg, u1a[...], preferred_element_type=F32))
        h = _ln(jnp.maximum(h + c1[...], 0.0), g[...], be[...])
        out_ref[...] = (jnp.dot(h, u2[...], preferred_element_type=F32)
                        + c2[...])

    return pl.pallas_call(
        body, grid=grid, in_specs=in_specs,
        out_specs=pl.BlockSpec((BN, T), lambda i: (i, 0)),
        out_shape=jax.ShapeDtypeStruct((N, T), F32),
    )(x, s_stk, cnt, *wlist)


def _prep_edge(p_edge, p_node1, F, H):
    w1 = p_edge["w1"]
    v1 = p_node1["w1"]
    return {
        "w1sT": w1[:, :F].T, "w1dT": w1[:, F:2 * F].T,
        "w1eT": w1[:, 2 * F:].T,
        "b1": p_edge["b1"][None, :], "g1": p_edge["g"][None, :],
        "be1": p_edge["be"][None, :], "w2T": p_edge["w2"].T,
        "b2": p_edge["b2"][None, :],
        "v1dT": v1[:, :F].T, "v1eT": v1[:, F:].T,
        "c1": p_node1["b1"][None, :], "g2": p_node1["g"][None, :],
        "be2": p_node1["be"][None, :], "v2T": p_node1["w2"].T,
        "c2": p_node1["b2"][None, :],
    }


def _prep_node2(p, F, H):
    u1 = p["w1"]
    return {
        "u1xT": u1[:, :F].T, "u1aT": u1[:, F:].T,
        "c1": p["b1"][None, :],
        "g": p["g"][None, :], "be": p["be"][None, :],
        "u2T": p["w2"].T,
        "c2": p["b2"][None, :],
    }


def _impl(x, edge_idx, edge_attr, params):
    row = edge_idx[0]
    col = edge_idx[1]
    N = x.shape[0]
    cnt = None
    ea = edge_attr
    for lname in ("l1", "l2", "l3"):
        p = params[lname]
        F = x.shape[1]
        H = p["edge"]["w2"].shape[0]
        last = lname == "l3"
        packed = lname != "l1"
        ew = _prep_edge(p["edge"], p["node1"], F, H)
        nw = _prep_node2(p["node2"], F, H)

        xg = _pack_cols(x) if packed else x
        src, dst = _sc_gather(xg, row, col)
        if last:
            o_stk = _tc_edge(src, dst, ea, ew, store_ea=False,
                             packed=packed)
            ea_next = None
        else:
            ea_next, o_stk = _tc_edge(src, dst, ea, ew, store_ea=True,
                                      packed=packed)
        if cnt is None:
            cnt = _sc_counts(row, N)
        s_stk = _sc_scatter(o_stk, row, N)
        x = _tc_node(x, s_stk, cnt, nw)
        ea = ea_next
    return x


kernel = jax.jit(_impl)
